# baseline (speedup 1.0000x reference)
import functools

import jax
import jax.numpy as jnp
from jax import lax
from jax.experimental import pallas as pl
from jax.experimental.pallas import tpu as pltpu
from jax.experimental.pallas import tpu_sc as plsc

N_PTS = 100000
N1 = 50000
E1 = 800000
E_PAD = 819200
D = 64
D2 = 128
N_CLASSES = 20

NCORE = 2
NSUB = 16
NW = NCORE * NSUB

_GCH = 800
_KCH = 400
_NQ = 6250
_NQP = 6272
_MCH = 4096
_MSTEPS = E_PAD // _MCH
_N1P = 51200

_SC_MESH = functools.partial(
    plsc.VectorSubcoreMesh, core_axis_name="c", subcore_axis_name="s")
_SC_PARAMS = pltpu.CompilerParams(needs_layout_passes=False)


def _mm_bias_body(x_ref, w_ref, b_ref, o_ref):
    o_ref[...] = (
        jnp.dot(x_ref[...], w_ref[...], preferred_element_type=jnp.float32)
        + b_ref[...])


def _tc_matmul_bias(x, w, b, blk):
    n, k = x.shape
    f = w.shape[1]
    return pl.pallas_call(
        _mm_bias_body,
        grid=(n // blk,),
        in_specs=[
            pl.BlockSpec((blk, k), lambda i: (i, 0)),
            pl.BlockSpec((k, f), lambda i: (0, 0)),
            pl.BlockSpec((1, f), lambda i: (0, 0)),
        ],
        out_specs=pl.BlockSpec((blk, f), lambda i: (i, 0)),
        out_shape=jax.ShapeDtypeStruct((n, f), jnp.float32),
    )(x, w, b)


def _prep_body(x_ref, cw_ref, w_ref, b_ref, a_ref, m_ref):
    cw = cw_ref[...]
    z = jnp.zeros_like(cw)
    a = (jnp.dot(x_ref[...], w_ref[...], preferred_element_type=jnp.float32)
         + cw + b_ref[...])
    a_ref[...] = jnp.concatenate([a, z], axis=1)
    m_ref[...] = jnp.concatenate([z, -cw], axis=1)


def _tc_prep(x, cw, w1a, b1, blk=2000):
    return pl.pallas_call(
        _prep_body,
        grid=(N1 // blk,),
        in_specs=[
            pl.BlockSpec((blk, D), lambda i: (i, 0)),
            pl.BlockSpec((blk, D), lambda i: (i, 0)),
            pl.BlockSpec((D, D), lambda i: (0, 0)),
            pl.BlockSpec((1, D), lambda i: (0, 0)),
        ],
        out_specs=[
            pl.BlockSpec((blk, D2), lambda i: (i, 0)),
            pl.BlockSpec((blk, D2), lambda i: (i, 0)),
        ],
        out_shape=[
            jax.ShapeDtypeStruct((N1, D2), jnp.float32),
            jax.ShapeDtypeStruct((N1, D2), jnp.float32),
        ],
    )(x, cw, w1a, b1)


def _d7_body(x_ref, w_ref, c_ref, o_ref):
    d = jnp.dot(x_ref[...], w_ref[...], preferred_element_type=jnp.float32)
    o_ref[...] = c_ref[...] + jnp.concatenate([jnp.zeros_like(d), d], axis=1)


def _tc_d7(x, w, c128, blk=2000):
    return pl.pallas_call(
        _d7_body,
        grid=(N1 // blk,),
        in_specs=[
            pl.BlockSpec((blk, D), lambda i: (i, 0)),
            pl.BlockSpec((D, D), lambda i: (0, 0)),
            pl.BlockSpec((blk, D2), lambda i: (i, 0)),
        ],
        out_specs=pl.BlockSpec((blk, D2), lambda i: (i, 0)),
        out_shape=jax.ShapeDtypeStruct((N1, D2), jnp.float32),
    )(x, w, c128)


def _pf_body(pre_ref, w2_ref, b2_ref, o_ref):
    h = jnp.maximum(pre_ref[:, 0:D], 0.0)
    pf = jnp.maximum(
        jnp.dot(h, w2_ref[...], preferred_element_type=jnp.float32)
        + b2_ref[...], 0.0)
    o_ref[...] = jnp.concatenate([pf, jnp.zeros_like(pf)], axis=1)


def _tc_point_mlp(pre128, w2, b2, blk=2000):
    return pl.pallas_call(
        _pf_body,
        grid=(N_PTS // blk,),
        in_specs=[
            pl.BlockSpec((blk, D2), lambda i: (i, 0)),
            pl.BlockSpec((D, D), lambda i: (0, 0)),
            pl.BlockSpec((1, D), lambda i: (0, 0)),
        ],
        out_specs=pl.BlockSpec((blk, D2), lambda i: (i, 0)),
        out_shape=jax.ShapeDtypeStruct((N_PTS, D2), jnp.float32),
    )(pre128, w2, b2)


def _edge_act_body(pre_ref, w2_ref, b2_ref, o_ref):
    pre = pre_ref[...]
    h = jnp.maximum(pre[:, 0:D] + pre[:, D:D2], 0.0)
    ef = lax.dot_general(w2_ref[...], h, (((0,), (1,)), ((), ())),
                         preferred_element_type=jnp.float32)
    ef = jnp.maximum(ef + b2_ref[...], 0.0)
    o_ref[0] = ef.reshape(D, _MCH // 128, 128)


def _tc_edge_act(pre128, w2, b2col):
    return pl.pallas_call(
        _edge_act_body,
        grid=(_MSTEPS,),
        in_specs=[
            pl.BlockSpec((_MCH, D2), lambda i: (i, 0)),
            pl.BlockSpec((D, D), lambda i: (0, 0)),
            pl.BlockSpec((D, 1), lambda i: (0, 0)),
        ],
        out_specs=pl.BlockSpec((1, D, _MCH // 128, 128),
                               lambda i: (i, 0, 0, 0)),
        out_shape=jax.ShapeDtypeStruct((_MSTEPS, D, _MCH // 128, 128),
                                       jnp.float32),
    )(pre128, w2, b2col)


def _out_mlp_body(aggt_ref, w1_ref, b1_ref, w2_ref, b2_ref, o_ref):
    aggt = aggt_ref[...].reshape(D, -1)
    h = jnp.maximum(
        lax.dot_general(w1_ref[...], aggt, (((0,), (0,)), ((), ())),
                        preferred_element_type=jnp.float32) + b1_ref[...],
        0.0)
    o_ref[...] = jnp.maximum(
        lax.dot_general(h, w2_ref[...], (((0,), (0,)), ((), ())),
                        preferred_element_type=jnp.float32) + b2_ref[...],
        0.0)


def _out_mlp_res_body(aggt_ref, w1_ref, b1_ref, w2_ref, b2_ref, res_ref,
                      o_ref):
    aggt = aggt_ref[...].reshape(D, -1)
    h = jnp.maximum(
        lax.dot_general(w1_ref[...], aggt, (((0,), (0,)), ((), ())),
                        preferred_element_type=jnp.float32) + b1_ref[...],
        0.0)
    o_ref[...] = jnp.maximum(
        lax.dot_general(h, w2_ref[...], (((0,), (0,)), ((), ())),
                        preferred_element_type=jnp.float32) + b2_ref[...],
        0.0) + res_ref[...]


def _tc_out_mlp(agg4, w1, b1col, w2, b2row, res=None, blk=2048):
    body = _out_mlp_body if res is None else _out_mlp_res_body
    in_specs = [
        pl.BlockSpec((D, blk // 128, 128), lambda i: (0, i, 0)),
        pl.BlockSpec((D, D), lambda i: (0, 0)),
        pl.BlockSpec((D, 1), lambda i: (0, 0)),
        pl.BlockSpec((D, D), lambda i: (0, 0)),
        pl.BlockSpec((1, D), lambda i: (0, 0)),
    ]
    args = [agg4, w1, b1col, w2, b2row]
    if res is not None:
        in_specs.append(pl.BlockSpec((blk, D), lambda i: (i, 0)))
        args.append(res)
    return pl.pallas_call(
        body,
        grid=(_N1P // blk,),
        in_specs=in_specs,
        out_specs=pl.BlockSpec((blk, D), lambda i: (i, 0)),
        out_shape=jax.ShapeDtypeStruct((_N1P, D), jnp.float32),
    )(*args)


def _final_body(pre_ref, w2_ref, b2_ref, cw_ref, cb_ref, o_ref):
    h = jnp.maximum(pre_ref[:, D:D2], 0.0)
    h2 = jnp.maximum(
        jnp.dot(h, w2_ref[...], preferred_element_type=jnp.float32)
        + b2_ref[...], 0.0)
    o_ref[...] = (
        jnp.dot(h2, cw_ref[...], preferred_element_type=jnp.float32)
        + cb_ref[...])


def _tc_final(pre128, w2, b2, cls_w, cls_b_row, blk=2000):
    return pl.pallas_call(
        _final_body,
        grid=(N_PTS // blk,),
        in_specs=[
            pl.BlockSpec((blk, D2), lambda i: (i, 0)),
            pl.BlockSpec((D, D), lambda i: (0, 0)),
            pl.BlockSpec((1, D), lambda i: (0, 0)),
            pl.BlockSpec((D, N_CLASSES), lambda i: (0, 0)),
            pl.BlockSpec((1, N_CLASSES), lambda i: (0, 0)),
        ],
        out_specs=pl.BlockSpec((blk, N_CLASSES), lambda i: (i, 0)),
        out_shape=jax.ShapeDtypeStruct((N_PTS, N_CLASSES), jnp.float32),
    )(pre128, w2, b2, cls_w, cls_b_row)


def _sc_gather_combine(q128, table128, idx):
    nchunks = N_PTS // _GCH

    @functools.partial(
        pl.kernel,
        out_type=jax.ShapeDtypeStruct((N_PTS, D2), jnp.float32),
        mesh=_SC_MESH(),
        compiler_params=_SC_PARAMS,
        scratch_types=[
            pltpu.VMEM((_GCH,), jnp.int32),
            pltpu.VMEM((_GCH, D2), jnp.float32),
        ],
    )
    def k(q_hbm, t_hbm, idx_hbm, out_hbm, idx_v, rows_v):
        wid = lax.axis_index("s") * NCORE + lax.axis_index("c")

        def body(j, carry):
            g = j * NW + wid

            @pl.when(g < nchunks)
            def _():
                pltpu.sync_copy(idx_hbm.at[pl.ds(g * _GCH, _GCH)], idx_v)
                pltpu.sync_copy(q_hbm.at[pl.ds(g * _GCH, _GCH)], rows_v)
                pltpu.sync_copy(t_hbm.at[idx_v], rows_v, add=True)
                pltpu.sync_copy(rows_v, out_hbm.at[pl.ds(g * _GCH, _GCH)])

            return carry

        lax.fori_loop(0, (nchunks + NW - 1) // NW, body, 0)

    return k(q128, table128, idx)


def _sc_two_gather(a128, m128, src, dst):
    gch = 400
    steps = E_PAD // gch // NW

    @functools.partial(
        pl.kernel,
        out_type=jax.ShapeDtypeStruct((E_PAD, D2), jnp.float32),
        mesh=_SC_MESH(),
        scratch_types=[
            pltpu.VMEM((gch,), jnp.int32),
            pltpu.VMEM((gch,), jnp.int32),
            pltpu.VMEM((gch,), jnp.int32),
            pltpu.VMEM((gch,), jnp.int32),
            pltpu.VMEM((gch, D2), jnp.float32),
            pltpu.VMEM((gch, D2), jnp.float32),
            pltpu.SemaphoreType.DMA,
            pltpu.SemaphoreType.DMA,
            pltpu.SemaphoreType.DMA,
            pltpu.SemaphoreType.DMA,
        ],
    )
    def k(a_hbm, m_hbm, src_hbm, dst_hbm, out_hbm, is0, is1, id0, id1,
          rows0, rows1, sa0, sa1, so0, so1):
        wid = lax.axis_index("s") * NCORE + lax.axis_index("c")

        def idx_in(j, isv, idv):
            g = j * NW + wid
            pltpu.sync_copy(src_hbm.at[pl.ds(g * gch, gch)], isv)
            pltpu.sync_copy(dst_hbm.at[pl.ds(g * gch, gch)], idv)

        def gather_start(isv, rows, sa):
            pltpu.async_copy(a_hbm.at[isv], rows, sa)

        def gather_wait(isv, rows, sa):
            pltpu.make_async_copy(a_hbm.at[isv], rows, sa).wait()

        def out_start(j, rows, so):
            g = j * NW + wid
            pltpu.async_copy(rows, out_hbm.at[pl.ds(g * gch, gch)], so)

        def out_wait(j, rows, so):
            g = j * NW + wid
            pltpu.make_async_copy(
                rows, out_hbm.at[pl.ds(g * gch, gch)], so).wait()

        idx_in(0, is0, id0)
        gather_start(is0, rows0, sa0)
        idx_in(1, is1, id1)
        gather_start(is1, rows1, sa1)

        def body(j, carry):
            b = lax.rem(j, 2)

            def finish(isv, idv, rows, sa, so):
                gather_wait(isv, rows, sa)
                pltpu.sync_copy(m_hbm.at[idv], rows, add=True)
                out_start(j, rows, so)

                @pl.when(j + 2 < steps)
                def _():
                    out_wait(j, rows, so)
                    idx_in(j + 2, isv, idv)
                    gather_start(isv, rows, sa)

            @pl.when(b == 0)
            def _():
                finish(is0, id0, rows0, sa0, so0)

            @pl.when(b == 1)
            def _():
                finish(is1, id1, rows1, sa1, so1)

            return carry

        lax.fori_loop(0, steps, body, 0)
        out_wait(steps - 2, rows0, so0)
        out_wait(steps - 1, rows1, so1)

    return k(a128, m128, src, dst)


def _sc_scatter_add(pf128, labels):
    nchunks = N_PTS // _KCH
    tile_rows = _NQP // NSUB

    @functools.partial(
        pl.kernel,
        out_type=jax.ShapeDtypeStruct((8, _NQP, D2), jnp.float32),
        mesh=_SC_MESH(),
        compiler_params=_SC_PARAMS,
        scratch_types=[
            pltpu.VMEM((_KCH,), jnp.int32),
            pltpu.VMEM((_KCH,), jnp.int32),
            pltpu.VMEM((_KCH, D2), jnp.float32),
            pltpu.VMEM((56, D2), jnp.float32),
            pltpu.VMEM_SHARED((_NQP, D2), jnp.float32),
        ],
    )
    def k(pf_hbm, lab_hbm, out_hbm, idx_v, fidx_v, rows_v, zbuf, acc):
        c = lax.axis_index("c")
        sid = lax.axis_index("s")

        def zrow(i, carry):
            def zlane(r, carry2):
                zbuf[i, pl.ds(r * 16, 16)] = jnp.zeros((16,), jnp.float32)
                return carry2

            return lax.fori_loop(0, 8, zlane, carry)

        lax.fori_loop(0, 56, zrow, 0)

        def octant(qi, carry):
            q = c * 4 + qi
            qbase = q * _NQ

            def zcp(j, carry2):
                pltpu.sync_copy(
                    zbuf, acc.at[pl.ds(sid * tile_rows + j * 56, 56)])
                return carry2

            lax.fori_loop(0, tile_rows // 56, zcp, 0)
            plsc.subcore_barrier()

            def body(j, carry2):
                g = j * NSUB + sid

                @pl.when(g < nchunks)
                def _():
                    pltpu.sync_copy(lab_hbm.at[pl.ds(g * _KCH, _KCH)],
                                    idx_v)

                    def remap(t, carry3):
                        iv = idx_v[pl.ds(t * 16, 16)]
                        lidx = iv - qbase
                        inq = (iv >= qbase) & (lidx < _NQ)
                        dummy = jnp.full((16,), _NQ, jnp.int32) + (t & 31)
                        fidx_v[pl.ds(t * 16, 16)] = jnp.where(
                            inq, lidx, dummy)
                        return carry3

                    lax.fori_loop(0, _KCH // 16, remap, 0)
                    pltpu.sync_copy(pf_hbm.at[pl.ds(g * _KCH, _KCH)],
                                    rows_v)
                    pltpu.sync_copy(rows_v, acc.at[fidx_v], add=True)

                return carry2

            lax.fori_loop(0, (nchunks + NSUB - 1) // NSUB, body, 0)
            plsc.subcore_barrier()
            pltpu.sync_copy(
                acc.at[pl.ds(sid * tile_rows, tile_rows)],
                out_hbm.at[q, pl.ds(sid * tile_rows, tile_rows)])
            plsc.subcore_barrier()
            return carry

        lax.fori_loop(0, 4, octant, 0)

    return k(pf128, labels)


def _sc_scatter_max(ef4, dst):
    nrow = _MCH // 128
    ngrp = _MCH // 16

    @functools.partial(
        pl.kernel,
        out_type=jax.ShapeDtypeStruct((D, _N1P // 128, 128), jnp.float32),
        mesh=_SC_MESH(),
        compiler_params=_SC_PARAMS,
        scratch_types=[
            pltpu.VMEM((_MCH,), jnp.int32),
            pltpu.VMEM((nrow, 128), jnp.float32),
            pltpu.VMEM((nrow, 128), jnp.float32),
            pltpu.VMEM((_N1P,), jnp.float32),
            pltpu.VMEM((_N1P,), jnp.float32),
            pltpu.VMEM((8, 128), jnp.float32),
        ],
    )
    def k(ef_hbm, dst_hbm, out_hbm, idx_v, v0_v, v1_v, acc0, acc1, stage):
        wid = lax.axis_index("s") * NCORE + lax.axis_index("c")
        col0 = wid * 2

        def zrow(i, carry):
            acc0[pl.ds(i * 16, 16)] = jnp.zeros((16,), jnp.float32)
            acc1[pl.ds(i * 16, 16)] = jnp.zeros((16,), jnp.float32)
            return carry

        lax.fori_loop(0, _N1P // 16, zrow, 0)

        def step(j, carry):
            pltpu.sync_copy(dst_hbm.at[pl.ds(j * _MCH, _MCH)], idx_v)
            pltpu.sync_copy(ef_hbm.at[j, col0], v0_v)
            pltpu.sync_copy(ef_hbm.at[j, col0 + 1], v1_v)

            def grp(g, carry2):
                iv = idx_v[pl.ds(g * 16, 16)]
                r = lax.shift_right_logical(g, 3)
                o = lax.bitwise_and(g, 7) * 16
                v0 = v0_v[r, pl.ds(o, 16)]
                v1 = v1_v[r, pl.ds(o, 16)]
                g0 = plsc.load_gather(acc0, (iv,))
                plsc.store_scatter(acc0, (iv,), jnp.maximum(g0, v0))
                g1 = plsc.load_gather(acc1, (iv,))
                plsc.store_scatter(acc1, (iv,), jnp.maximum(g1, v1))
                return carry2

            lax.fori_loop(0, ngrp, grp, 0)

            def vgrp(g, confl):
                iv = idx_v[pl.ds(g * 16, 16)]
                r = lax.shift_right_logical(g, 3)
                o = lax.bitwise_and(g, 7) * 16
                v0 = v0_v[r, pl.ds(o, 16)]
                v1 = v1_v[r, pl.ds(o, 16)]
                r0 = plsc.load_gather(acc0, (iv,))
                r1 = plsc.load_gather(acc1, (iv,))
                return confl | (v0 > r0) | (v1 > r1)

            confl = lax.fori_loop(0, ngrp, vgrp,
                                  jnp.zeros((16,), jnp.bool_))
            cnt = jnp.max(plsc.all_reduce_population_count(confl))

            def fix_round(c):
                def grp2(g, confl2):
                    iv = idx_v[pl.ds(g * 16, 16)]
                    r = lax.shift_right_logical(g, 3)
                    o = lax.bitwise_and(g, 7) * 16
                    v0 = v0_v[r, pl.ds(o, 16)]
                    v1 = v1_v[r, pl.ds(o, 16)]
                    r0 = plsc.load_gather(acc0, (iv,))
                    plsc.store_scatter(acc0, (iv,), v0, mask=v0 > r0)
                    r1 = plsc.load_gather(acc1, (iv,))
                    plsc.store_scatter(acc1, (iv,), v1, mask=v1 > r1)
                    a0 = plsc.load_gather(acc0, (iv,))
                    a1 = plsc.load_gather(acc1, (iv,))
                    return confl2 | (v0 > a0) | (v1 > a1)

                c2 = lax.fori_loop(0, ngrp, grp2,
                                   jnp.zeros((16,), jnp.bool_))
                return jnp.max(plsc.all_reduce_population_count(c2))

            lax.while_loop(lambda c: c > 0, fix_round, cnt)
            return carry

        lax.fori_loop(0, _MSTEPS, step, 0)

        def wcol(col, acc):
            def wchunk(j, carry):
                def wrow(t, carry2):
                    r = lax.shift_right_logical(t, 3)
                    o = lax.bitwise_and(t, 7) * 16
                    stage[r, pl.ds(o, 16)] = acc[
                        pl.ds(j * 1024 + r * 128 + o, 16)]
                    return carry2

                lax.fori_loop(0, 64, wrow, 0)
                pltpu.sync_copy(stage, out_hbm.at[col, pl.ds(j * 8, 8)])
                return carry

            lax.fori_loop(0, _N1P // 1024, wchunk, 0)

        wcol(col0, acc0)
        wcol(col0 + 1, acc1)

    return k(ef4, dst)


def kernel(remission, points, l1_cluster_centers, l2_cluster_centers,
           l1_edges, l2_edges, l1_labels, l2_labels,
           l1_ffn, l2_edge_mlp, l2_out_mlp, l6_edge_mlp, l6_out_mlp,
           l7_fbn, cls_W, cls_b):
    del l2_cluster_centers, l2_edges, l2_labels
    f32 = jnp.float32
    centers = l1_cluster_centers.astype(f32)
    labels = l1_labels.astype(jnp.int32)
    src = l1_edges[:, 0].astype(jnp.int32)
    dst = l1_edges[:, 1].astype(jnp.int32)
    src_g = jnp.pad(src, (0, E_PAD - E1))
    dst_g = jnp.pad(dst, (0, E_PAD - E1))
    dst_s = jnp.pad(dst, (0, E_PAD - E1), constant_values=N1)

    w1_1, b1_1, w2_1, b2_1 = l1_ffn
    w1_2, b1_2, w2_2, b2_2 = l2_edge_mlp
    w1_6, b1_6, w2_6, b2_6 = l6_edge_mlp
    w1_7, b1_7, w2_7, b2_7 = l7_fbn
    w1p = w1_1[1:4]
    w1r = w1_1[0:1]
    w2b = w1_2[D:D + 3]
    w6b = w1_6[D:D + 3]
    w7a = w1_7[0:D]
    w7b = w1_7[D:D + 3]

    z64 = jnp.zeros((3, D), f32)
    wc = jnp.concatenate([-w1p, z64, w2b, w6b, z64, -w7b], axis=1)
    wc4 = jnp.pad(wc, ((0, 1), (0, 0)))
    centers4 = jnp.pad(centers, ((0, 0), (0, 1)))
    cw_all = _tc_matmul_bias(centers4, wc4, jnp.zeros((1, 6 * D), f32),
                             blk=2000)
    tab1 = cw_all[:, 0:D2]
    cw2 = cw_all[:, D2:D2 + D]
    cw6 = cw_all[:, D2 + D:D2 + 2 * D]
    tab7b = cw_all[:, D2 + 2 * D:D2 + 2 * D + D2]

    pts4 = jnp.concatenate([remission.astype(f32), points.astype(f32)],
                           axis=1)
    w4 = jnp.concatenate([
        jnp.concatenate([w1r, w1p], axis=0),
        jnp.concatenate([jnp.zeros((1, D), f32), w7b], axis=0),
    ], axis=1)
    bq = jnp.concatenate([b1_1, b1_7])[None, :]
    q_cat = _tc_matmul_bias(pts4, w4, bq, blk=2000)

    pre1 = _sc_gather_combine(q_cat, tab1, labels)
    pf128 = _tc_point_mlp(pre1, w2_1, b2_1[None, :])
    t18 = _sc_scatter_add(pf128, labels)
    t1 = t18[:, :_NQ, :D].reshape(N1, D)

    a2, m2 = _tc_prep(t1, cw2, w1_2[0:D], b1_2[None, :])
    pre2 = _sc_two_gather(a2, m2, src_g, dst_g)
    eft2 = _tc_edge_act(pre2, w2_2, b2_2[:, None])
    agg2 = _sc_scatter_max(eft2, dst_s)
    w1o2, b1o2, w2o2, b2o2 = l2_out_mlp
    t2p = _tc_out_mlp(agg2, w1o2, b1o2[:, None], w2o2, b2o2[None, :])
    t2 = t2p[:N1]

    a6, m6 = _tc_prep(t2, cw6, w1_6[0:D], b1_6[None, :])
    pre6 = _sc_two_gather(a6, m6, src_g, dst_g)
    eft6 = _tc_edge_act(pre6, w2_6, b2_6[:, None])
    agg6 = _sc_scatter_max(eft6, dst_s)
    w1o6, b1o6, w2o6, b2o6 = l6_out_mlp
    t6p = _tc_out_mlp(agg6, w1o6, b1o6[:, None], w2o6, b2o6[None, :],
                      res=t2p)
    t6 = t6p[:N1]

    d7 = _tc_d7(t6, w7a, tab7b)
    pre7 = _sc_gather_combine(q_cat, d7, labels)
    return _tc_final(pre7, w2_7, b2_7[None, :], cls_W.astype(f32),
                     cls_b.astype(f32)[None, :])


# speedup vs baseline: 1.4489x; 1.4489x over previous
import functools

import jax
import jax.numpy as jnp
from jax import lax
from jax.experimental import pallas as pl
from jax.experimental.pallas import tpu as pltpu
from jax.experimental.pallas import tpu_sc as plsc

N_PTS = 100000
N1 = 50000
E1 = 800000
E_PAD = 819200
D = 64
D2 = 128
N_CLASSES = 20

NCORE = 2
NSUB = 16
NW = NCORE * NSUB

_GCH = 800
_KCH = 400
_NQ = 6250
_NQP = 6272
_MCH = 4096
_MSTEPS = E_PAD // _MCH
_N1P = 51200

_SC_MESH = functools.partial(
    plsc.VectorSubcoreMesh, core_axis_name="c", subcore_axis_name="s")
_SC_PARAMS = pltpu.CompilerParams(needs_layout_passes=False)


def _mm_bias_body(x_ref, w_ref, b_ref, o_ref):
    o_ref[...] = (
        jnp.dot(x_ref[...], w_ref[...], preferred_element_type=jnp.float32)
        + b_ref[...])


def _tc_matmul_bias(x, w, b, blk):
    n, k = x.shape
    f = w.shape[1]
    return pl.pallas_call(
        _mm_bias_body,
        grid=(n // blk,),
        in_specs=[
            pl.BlockSpec((blk, k), lambda i: (i, 0)),
            pl.BlockSpec((k, f), lambda i: (0, 0)),
            pl.BlockSpec((1, f), lambda i: (0, 0)),
        ],
        out_specs=pl.BlockSpec((blk, f), lambda i: (i, 0)),
        out_shape=jax.ShapeDtypeStruct((n, f), jnp.float32),
    )(x, w, b)


def _prep_body(x_ref, cw_ref, w_ref, b_ref, a_ref, m_ref):
    cw = cw_ref[...]
    z = jnp.zeros_like(cw)
    a = (jnp.dot(x_ref[...], w_ref[...], preferred_element_type=jnp.float32)
         + cw + b_ref[...])
    a_ref[...] = jnp.concatenate([a, z], axis=1)
    m_ref[...] = jnp.concatenate([z, -cw], axis=1)


def _tc_prep(x, cw, w1a, b1, blk=2000):
    return pl.pallas_call(
        _prep_body,
        grid=(N1 // blk,),
        in_specs=[
            pl.BlockSpec((blk, D), lambda i: (i, 0)),
            pl.BlockSpec((blk, D), lambda i: (i, 0)),
            pl.BlockSpec((D, D), lambda i: (0, 0)),
            pl.BlockSpec((1, D), lambda i: (0, 0)),
        ],
        out_specs=[
            pl.BlockSpec((blk, D2), lambda i: (i, 0)),
            pl.BlockSpec((blk, D2), lambda i: (i, 0)),
        ],
        out_shape=[
            jax.ShapeDtypeStruct((N1, D2), jnp.float32),
            jax.ShapeDtypeStruct((N1, D2), jnp.float32),
        ],
    )(x, cw, w1a, b1)


def _d7_body(x_ref, w_ref, c_ref, o_ref):
    d = jnp.dot(x_ref[...], w_ref[...], preferred_element_type=jnp.float32)
    o_ref[...] = c_ref[...] + jnp.concatenate([jnp.zeros_like(d), d], axis=1)


def _tc_d7(x, w, c128, blk=2000):
    return pl.pallas_call(
        _d7_body,
        grid=(N1 // blk,),
        in_specs=[
            pl.BlockSpec((blk, D), lambda i: (i, 0)),
            pl.BlockSpec((D, D), lambda i: (0, 0)),
            pl.BlockSpec((blk, D2), lambda i: (i, 0)),
        ],
        out_specs=pl.BlockSpec((blk, D2), lambda i: (i, 0)),
        out_shape=jax.ShapeDtypeStruct((N1, D2), jnp.float32),
    )(x, w, c128)


def _pf_body(pre_ref, w2_ref, b2_ref, o_ref):
    h = jnp.maximum(pre_ref[:, 0:D], 0.0)
    pf = jnp.maximum(
        jnp.dot(h, w2_ref[...], preferred_element_type=jnp.float32)
        + b2_ref[...], 0.0)
    o_ref[...] = jnp.concatenate([pf, jnp.zeros_like(pf)], axis=1)


def _tc_point_mlp(pre128, w2, b2, blk=2000):
    return pl.pallas_call(
        _pf_body,
        grid=(N_PTS // blk,),
        in_specs=[
            pl.BlockSpec((blk, D2), lambda i: (i, 0)),
            pl.BlockSpec((D, D), lambda i: (0, 0)),
            pl.BlockSpec((1, D), lambda i: (0, 0)),
        ],
        out_specs=pl.BlockSpec((blk, D2), lambda i: (i, 0)),
        out_shape=jax.ShapeDtypeStruct((N_PTS, D2), jnp.float32),
    )(pre128, w2, b2)


def _edge_act_body(pre_ref, w2_ref, b2_ref, o_ref):
    pre = pre_ref[...]
    h = jnp.maximum(pre[:, 0:D] + pre[:, D:D2], 0.0)
    ef = lax.dot_general(w2_ref[...], h, (((0,), (1,)), ((), ())),
                         preferred_element_type=jnp.float32)
    ef = jnp.maximum(ef + b2_ref[...], 0.0)
    o_ref[0] = ef.reshape(D, _MCH // 128, 128)


def _tc_edge_act(pre128, w2, b2col):
    return pl.pallas_call(
        _edge_act_body,
        grid=(_MSTEPS,),
        in_specs=[
            pl.BlockSpec((_MCH, D2), lambda i: (i, 0)),
            pl.BlockSpec((D, D), lambda i: (0, 0)),
            pl.BlockSpec((D, 1), lambda i: (0, 0)),
        ],
        out_specs=pl.BlockSpec((1, D, _MCH // 128, 128),
                               lambda i: (i, 0, 0, 0)),
        out_shape=jax.ShapeDtypeStruct((_MSTEPS, D, _MCH // 128, 128),
                                       jnp.float32),
    )(pre128, w2, b2col)


def _out_mlp_body(aggt_ref, w1_ref, b1_ref, w2_ref, b2_ref, o_ref):
    aggt = aggt_ref[...].reshape(D, -1)
    h = jnp.maximum(
        lax.dot_general(w1_ref[...], aggt, (((0,), (0,)), ((), ())),
                        preferred_element_type=jnp.float32) + b1_ref[...],
        0.0)
    o_ref[...] = jnp.maximum(
        lax.dot_general(h, w2_ref[...], (((0,), (0,)), ((), ())),
                        preferred_element_type=jnp.float32) + b2_ref[...],
        0.0)


def _out_mlp_res_body(aggt_ref, w1_ref, b1_ref, w2_ref, b2_ref, res_ref,
                      o_ref):
    aggt = aggt_ref[...].reshape(D, -1)
    h = jnp.maximum(
        lax.dot_general(w1_ref[...], aggt, (((0,), (0,)), ((), ())),
                        preferred_element_type=jnp.float32) + b1_ref[...],
        0.0)
    o_ref[...] = jnp.maximum(
        lax.dot_general(h, w2_ref[...], (((0,), (0,)), ((), ())),
                        preferred_element_type=jnp.float32) + b2_ref[...],
        0.0) + res_ref[...]


def _tc_out_mlp(agg4, w1, b1col, w2, b2row, res=None, blk=2048):
    body = _out_mlp_body if res is None else _out_mlp_res_body
    in_specs = [
        pl.BlockSpec((D, blk // 128, 128), lambda i: (0, i, 0)),
        pl.BlockSpec((D, D), lambda i: (0, 0)),
        pl.BlockSpec((D, 1), lambda i: (0, 0)),
        pl.BlockSpec((D, D), lambda i: (0, 0)),
        pl.BlockSpec((1, D), lambda i: (0, 0)),
    ]
    args = [agg4, w1, b1col, w2, b2row]
    if res is not None:
        in_specs.append(pl.BlockSpec((blk, D), lambda i: (i, 0)))
        args.append(res)
    return pl.pallas_call(
        body,
        grid=(_N1P // blk,),
        in_specs=in_specs,
        out_specs=pl.BlockSpec((blk, D), lambda i: (i, 0)),
        out_shape=jax.ShapeDtypeStruct((_N1P, D), jnp.float32),
    )(*args)


def _final_body(pre_ref, w2_ref, b2_ref, cw_ref, cb_ref, o_ref):
    h = jnp.maximum(pre_ref[:, D:D2], 0.0)
    h2 = jnp.maximum(
        jnp.dot(h, w2_ref[...], preferred_element_type=jnp.float32)
        + b2_ref[...], 0.0)
    o_ref[...] = (
        jnp.dot(h2, cw_ref[...], preferred_element_type=jnp.float32)
        + cb_ref[...])


def _tc_final(pre128, w2, b2, cls_w, cls_b_row, blk=2000):
    return pl.pallas_call(
        _final_body,
        grid=(N_PTS // blk,),
        in_specs=[
            pl.BlockSpec((blk, D2), lambda i: (i, 0)),
            pl.BlockSpec((D, D), lambda i: (0, 0)),
            pl.BlockSpec((1, D), lambda i: (0, 0)),
            pl.BlockSpec((D, N_CLASSES), lambda i: (0, 0)),
            pl.BlockSpec((1, N_CLASSES), lambda i: (0, 0)),
        ],
        out_specs=pl.BlockSpec((blk, N_CLASSES), lambda i: (i, 0)),
        out_shape=jax.ShapeDtypeStruct((N_PTS, N_CLASSES), jnp.float32),
    )(pre128, w2, b2, cls_w, cls_b_row)


def _sc_gather_combine(q128, table128, idx):
    nchunks = N_PTS // _GCH

    @functools.partial(
        pl.kernel,
        out_type=jax.ShapeDtypeStruct((N_PTS, D2), jnp.float32),
        mesh=_SC_MESH(),
        compiler_params=_SC_PARAMS,
        scratch_types=[
            pltpu.VMEM((_GCH,), jnp.int32),
            pltpu.VMEM((_GCH, D2), jnp.float32),
        ],
    )
    def k(q_hbm, t_hbm, idx_hbm, out_hbm, idx_v, rows_v):
        wid = lax.axis_index("s") * NCORE + lax.axis_index("c")

        def body(j, carry):
            g = j * NW + wid

            @pl.when(g < nchunks)
            def _():
                pltpu.sync_copy(idx_hbm.at[pl.ds(g * _GCH, _GCH)], idx_v)
                pltpu.sync_copy(q_hbm.at[pl.ds(g * _GCH, _GCH)], rows_v)
                pltpu.sync_copy(t_hbm.at[idx_v], rows_v, add=True)
                pltpu.sync_copy(rows_v, out_hbm.at[pl.ds(g * _GCH, _GCH)])

            return carry

        lax.fori_loop(0, (nchunks + NW - 1) // NW, body, 0)

    return k(q128, table128, idx)


def _sc_two_gather(a128, m128, src, dst):
    gch = 400
    steps = E_PAD // gch // NW

    @functools.partial(
        pl.kernel,
        out_type=jax.ShapeDtypeStruct((E_PAD, D2), jnp.float32),
        mesh=_SC_MESH(),
        scratch_types=[
            pltpu.VMEM((gch,), jnp.int32),
            pltpu.VMEM((gch,), jnp.int32),
            pltpu.VMEM((gch,), jnp.int32),
            pltpu.VMEM((gch,), jnp.int32),
            pltpu.VMEM((gch, D2), jnp.float32),
            pltpu.VMEM((gch, D2), jnp.float32),
            pltpu.SemaphoreType.DMA,
            pltpu.SemaphoreType.DMA,
            pltpu.SemaphoreType.DMA,
            pltpu.SemaphoreType.DMA,
        ],
    )
    def k(a_hbm, m_hbm, src_hbm, dst_hbm, out_hbm, is0, is1, id0, id1,
          rows0, rows1, sa0, sa1, so0, so1):
        wid = lax.axis_index("s") * NCORE + lax.axis_index("c")

        def idx_in(j, isv, idv):
            g = j * NW + wid
            pltpu.sync_copy(src_hbm.at[pl.ds(g * gch, gch)], isv)
            pltpu.sync_copy(dst_hbm.at[pl.ds(g * gch, gch)], idv)

        def gather_start(isv, rows, sa):
            pltpu.async_copy(a_hbm.at[isv], rows, sa)

        def gather_wait(isv, rows, sa):
            pltpu.make_async_copy(a_hbm.at[isv], rows, sa).wait()

        def out_start(j, rows, so):
            g = j * NW + wid
            pltpu.async_copy(rows, out_hbm.at[pl.ds(g * gch, gch)], so)

        def out_wait(j, rows, so):
            g = j * NW + wid
            pltpu.make_async_copy(
                rows, out_hbm.at[pl.ds(g * gch, gch)], so).wait()

        idx_in(0, is0, id0)
        gather_start(is0, rows0, sa0)
        idx_in(1, is1, id1)
        gather_start(is1, rows1, sa1)

        def body(j, carry):
            b = lax.rem(j, 2)

            def finish(isv, idv, rows, sa, so):
                gather_wait(isv, rows, sa)
                pltpu.sync_copy(m_hbm.at[idv], rows, add=True)
                out_start(j, rows, so)

                @pl.when(j + 2 < steps)
                def _():
                    out_wait(j, rows, so)
                    idx_in(j + 2, isv, idv)
                    gather_start(isv, rows, sa)

            @pl.when(b == 0)
            def _():
                finish(is0, id0, rows0, sa0, so0)

            @pl.when(b == 1)
            def _():
                finish(is1, id1, rows1, sa1, so1)

            return carry

        lax.fori_loop(0, steps, body, 0)
        out_wait(steps - 2, rows0, so0)
        out_wait(steps - 1, rows1, so1)

    return k(a128, m128, src, dst)


def _sc_scatter_add(pf128, labels):
    nchunks = N_PTS // _KCH
    tile_rows = _NQP // NSUB

    @functools.partial(
        pl.kernel,
        out_type=jax.ShapeDtypeStruct((8, _NQP, D2), jnp.float32),
        mesh=_SC_MESH(),
        compiler_params=_SC_PARAMS,
        scratch_types=[
            pltpu.VMEM((_KCH,), jnp.int32),
            pltpu.VMEM((_KCH,), jnp.int32),
            pltpu.VMEM((_KCH, D2), jnp.float32),
            pltpu.VMEM((56, D2), jnp.float32),
            pltpu.VMEM_SHARED((_NQP, D2), jnp.float32),
        ],
    )
    def k(pf_hbm, lab_hbm, out_hbm, idx_v, fidx_v, rows_v, zbuf, acc):
        c = lax.axis_index("c")
        sid = lax.axis_index("s")

        def zrow(i, carry):
            def zlane(r, carry2):
                zbuf[i, pl.ds(r * 16, 16)] = jnp.zeros((16,), jnp.float32)
                return carry2

            return lax.fori_loop(0, 8, zlane, carry)

        lax.fori_loop(0, 56, zrow, 0)

        def octant(qi, carry):
            q = c * 4 + qi
            qbase = q * _NQ

            def zcp(j, carry2):
                pltpu.sync_copy(
                    zbuf, acc.at[pl.ds(sid * tile_rows + j * 56, 56)])
                return carry2

            lax.fori_loop(0, tile_rows // 56, zcp, 0)
            plsc.subcore_barrier()

            def body(j, carry2):
                g = j * NSUB + sid

                @pl.when(g < nchunks)
                def _():
                    pltpu.sync_copy(lab_hbm.at[pl.ds(g * _KCH, _KCH)],
                                    idx_v)

                    def remap(t, carry3):
                        iv = idx_v[pl.ds(t * 16, 16)]
                        lidx = iv - qbase
                        inq = (iv >= qbase) & (lidx < _NQ)
                        dummy = jnp.full((16,), _NQ, jnp.int32) + (t & 31)
                        fidx_v[pl.ds(t * 16, 16)] = jnp.where(
                            inq, lidx, dummy)
                        return carry3

                    lax.fori_loop(0, _KCH // 16, remap, 0)
                    pltpu.sync_copy(pf_hbm.at[pl.ds(g * _KCH, _KCH)],
                                    rows_v)
                    pltpu.sync_copy(rows_v, acc.at[fidx_v], add=True)

                return carry2

            lax.fori_loop(0, (nchunks + NSUB - 1) // NSUB, body, 0)
            plsc.subcore_barrier()
            pltpu.sync_copy(
                acc.at[pl.ds(sid * tile_rows, tile_rows)],
                out_hbm.at[q, pl.ds(sid * tile_rows, tile_rows)])
            plsc.subcore_barrier()
            return carry

        lax.fori_loop(0, 4, octant, 0)

    return k(pf128, labels)


def _sc_scatter_max(ef4, dst):
    nrow = _MCH // 128
    ngrp = _MCH // 16

    @functools.partial(
        pl.kernel,
        out_type=jax.ShapeDtypeStruct((D, _N1P // 128, 128), jnp.float32),
        mesh=_SC_MESH(),
        compiler_params=_SC_PARAMS,
        scratch_types=[
            pltpu.VMEM((_MCH,), jnp.int32),
            pltpu.VMEM((nrow, 128), jnp.float32),
            pltpu.VMEM((nrow, 128), jnp.float32),
            pltpu.VMEM((_N1P,), jnp.float32),
            pltpu.VMEM((_N1P,), jnp.float32),
            pltpu.VMEM((8, 128), jnp.float32),
        ],
    )
    def k(ef_hbm, dst_hbm, out_hbm, idx_v, v0_v, v1_v, acc0, acc1, stage):
        wid = lax.axis_index("s") * NCORE + lax.axis_index("c")
        col0 = wid * 2

        def zrow(i, carry):
            acc0[pl.ds(i * 16, 16)] = jnp.zeros((16,), jnp.float32)
            acc1[pl.ds(i * 16, 16)] = jnp.zeros((16,), jnp.float32)
            return carry

        lax.fori_loop(0, _N1P // 16, zrow, 0)

        def step(j, carry):
            pltpu.sync_copy(dst_hbm.at[pl.ds(j * _MCH, _MCH)], idx_v)
            pltpu.sync_copy(ef_hbm.at[j, col0], v0_v)
            pltpu.sync_copy(ef_hbm.at[j, col0 + 1], v1_v)

            def grp(g4, confl):
                for u in range(4):
                    g = g4 * 4 + u
                    iv = idx_v[pl.ds(g * 16, 16)]
                    r = lax.shift_right_logical(g, 3)
                    o = lax.bitwise_and(g, 7) * 16
                    v0 = v0_v[r, pl.ds(o, 16)]
                    v1 = v1_v[r, pl.ds(o, 16)]
                    g0 = plsc.load_gather(acc0, (iv,))
                    plsc.store_scatter(acc0, (iv,), jnp.maximum(g0, v0))
                    g1 = plsc.load_gather(acc1, (iv,))
                    plsc.store_scatter(acc1, (iv,), jnp.maximum(g1, v1))
                    r0 = plsc.load_gather(acc0, (iv,))
                    r1 = plsc.load_gather(acc1, (iv,))
                    confl = confl | (v0 > r0) | (v1 > r1)
                return confl

            confl = lax.fori_loop(0, ngrp // 4, grp,
                                  jnp.zeros((16,), jnp.bool_))
            cnt = jnp.max(plsc.all_reduce_population_count(confl))

            def fix_round(c):
                def grp2(g, confl2):
                    iv = idx_v[pl.ds(g * 16, 16)]
                    r = lax.shift_right_logical(g, 3)
                    o = lax.bitwise_and(g, 7) * 16
                    v0 = v0_v[r, pl.ds(o, 16)]
                    v1 = v1_v[r, pl.ds(o, 16)]
                    r0 = plsc.load_gather(acc0, (iv,))
                    plsc.store_scatter(acc0, (iv,), v0, mask=v0 > r0)
                    r1 = plsc.load_gather(acc1, (iv,))
                    plsc.store_scatter(acc1, (iv,), v1, mask=v1 > r1)
                    a0 = plsc.load_gather(acc0, (iv,))
                    a1 = plsc.load_gather(acc1, (iv,))
                    return confl2 | (v0 > a0) | (v1 > a1)

                c2 = lax.fori_loop(0, ngrp, grp2,
                                   jnp.zeros((16,), jnp.bool_))
                return jnp.max(plsc.all_reduce_population_count(c2))

            lax.while_loop(lambda c: c > 0, fix_round, cnt)
            return carry

        lax.fori_loop(0, _MSTEPS, step, 0)

        def wcol(col, acc):
            def wchunk(j, carry):
                def wrow(t, carry2):
                    r = lax.shift_right_logical(t, 3)
                    o = lax.bitwise_and(t, 7) * 16
                    stage[r, pl.ds(o, 16)] = acc[
                        pl.ds(j * 1024 + r * 128 + o, 16)]
                    return carry2

                lax.fori_loop(0, 64, wrow, 0)
                pltpu.sync_copy(stage, out_hbm.at[col, pl.ds(j * 8, 8)])
                return carry

            lax.fori_loop(0, _N1P // 1024, wchunk, 0)

        wcol(col0, acc0)
        wcol(col0 + 1, acc1)

    return k(ef4, dst)


def kernel(remission, points, l1_cluster_centers, l2_cluster_centers,
           l1_edges, l2_edges, l1_labels, l2_labels,
           l1_ffn, l2_edge_mlp, l2_out_mlp, l6_edge_mlp, l6_out_mlp,
           l7_fbn, cls_W, cls_b):
    del l2_cluster_centers, l2_edges, l2_labels
    f32 = jnp.float32
    centers = l1_cluster_centers.astype(f32)
    labels = l1_labels.astype(jnp.int32)
    src = l1_edges[:, 0].astype(jnp.int32)
    dst = l1_edges[:, 1].astype(jnp.int32)
    pad_idx = (jnp.arange(E_PAD - E1, dtype=jnp.int32) * 97) % N1
    src_g = jnp.concatenate([src, pad_idx])
    dst_g = jnp.concatenate([dst, pad_idx])
    dst_s = jnp.pad(dst, (0, E_PAD - E1), constant_values=N1)

    w1_1, b1_1, w2_1, b2_1 = l1_ffn
    w1_2, b1_2, w2_2, b2_2 = l2_edge_mlp
    w1_6, b1_6, w2_6, b2_6 = l6_edge_mlp
    w1_7, b1_7, w2_7, b2_7 = l7_fbn
    w1p = w1_1[1:4]
    w1r = w1_1[0:1]
    w2b = w1_2[D:D + 3]
    w6b = w1_6[D:D + 3]
    w7a = w1_7[0:D]
    w7b = w1_7[D:D + 3]

    z64 = jnp.zeros((3, D), f32)
    wc = jnp.concatenate([-w1p, z64, w2b, w6b, z64, -w7b], axis=1)
    wc4 = jnp.pad(wc, ((0, 1), (0, 0)))
    centers4 = jnp.pad(centers, ((0, 0), (0, 1)))
    cw_all = _tc_matmul_bias(centers4, wc4, jnp.zeros((1, 6 * D), f32),
                             blk=2000)
    tab1 = cw_all[:, 0:D2]
    cw2 = cw_all[:, D2:D2 + D]
    cw6 = cw_all[:, D2 + D:D2 + 2 * D]
    tab7b = cw_all[:, D2 + 2 * D:D2 + 2 * D + D2]

    pts4 = jnp.concatenate([remission.astype(f32), points.astype(f32)],
                           axis=1)
    w4 = jnp.concatenate([
        jnp.concatenate([w1r, w1p], axis=0),
        jnp.concatenate([jnp.zeros((1, D), f32), w7b], axis=0),
    ], axis=1)
    bq = jnp.concatenate([b1_1, b1_7])[None, :]
    q_cat = _tc_matmul_bias(pts4, w4, bq, blk=2000)

    pre1 = _sc_gather_combine(q_cat, tab1, labels)
    pf128 = _tc_point_mlp(pre1, w2_1, b2_1[None, :])
    t18 = _sc_scatter_add(pf128, labels)
    t1 = t18[:, :_NQ, :D].reshape(N1, D)

    a2, m2 = _tc_prep(t1, cw2, w1_2[0:D], b1_2[None, :])
    pre2 = _sc_two_gather(a2, m2, src_g, dst_g)
    eft2 = _tc_edge_act(pre2, w2_2, b2_2[:, None])
    agg2 = _sc_scatter_max(eft2, dst_s)
    w1o2, b1o2, w2o2, b2o2 = l2_out_mlp
    t2p = _tc_out_mlp(agg2, w1o2, b1o2[:, None], w2o2, b2o2[None, :])
    t2 = t2p[:N1]

    a6, m6 = _tc_prep(t2, cw6, w1_6[0:D], b1_6[None, :])
    pre6 = _sc_two_gather(a6, m6, src_g, dst_g)
    eft6 = _tc_edge_act(pre6, w2_6, b2_6[:, None])
    agg6 = _sc_scatter_max(eft6, dst_s)
    w1o6, b1o6, w2o6, b2o6 = l6_out_mlp
    t6p = _tc_out_mlp(agg6, w1o6, b1o6[:, None], w2o6, b2o6[None, :],
                      res=t2p)
    t6 = t6p[:N1]

    d7 = _tc_d7(t6, w7a, tab7b)
    pre7 = _sc_gather_combine(q_cat, d7, labels)
    return _tc_final(pre7, w2_7, b2_7[None, :], cls_W.astype(f32),
                     cls_b.astype(f32)[None, :])


# speedup vs baseline: 1.6442x; 1.1347x over previous
import functools

import jax
import jax.numpy as jnp
from jax import lax
from jax.experimental import pallas as pl
from jax.experimental.pallas import tpu as pltpu
from jax.experimental.pallas import tpu_sc as plsc

N_PTS = 100000
N1 = 50000
E1 = 800000
E_PAD = 819200
D = 64
D2 = 128
N_CLASSES = 20

NCORE = 2
NSUB = 16
NW = NCORE * NSUB

_GCH = 800
_KCH = 400
_NQ = 6250
_NQP = 6272
_MCH = 4096
_MSTEPS = E_PAD // _MCH
_N1P = 51200

_SC_MESH = functools.partial(
    plsc.VectorSubcoreMesh, core_axis_name="c", subcore_axis_name="s")
_SC_PARAMS = pltpu.CompilerParams(needs_layout_passes=False)


def _mm_bias_body(x_ref, w_ref, b_ref, o_ref):
    o_ref[...] = (
        jnp.dot(x_ref[...], w_ref[...], preferred_element_type=jnp.float32)
        + b_ref[...])


def _tc_matmul_bias(x, w, b, blk):
    n, k = x.shape
    f = w.shape[1]
    return pl.pallas_call(
        _mm_bias_body,
        grid=(n // blk,),
        in_specs=[
            pl.BlockSpec((blk, k), lambda i: (i, 0)),
            pl.BlockSpec((k, f), lambda i: (0, 0)),
            pl.BlockSpec((1, f), lambda i: (0, 0)),
        ],
        out_specs=pl.BlockSpec((blk, f), lambda i: (i, 0)),
        out_shape=jax.ShapeDtypeStruct((n, f), jnp.float32),
    )(x, w, b)


def _prep_body(x_ref, cw_ref, w_ref, b_ref, a_ref, m_ref):
    cw = cw_ref[...]
    z = jnp.zeros_like(cw)
    a = (jnp.dot(x_ref[...], w_ref[...], preferred_element_type=jnp.float32)
         + cw + b_ref[...])
    a_ref[...] = jnp.concatenate([a, z], axis=1)
    m_ref[...] = jnp.concatenate([z, -cw], axis=1)


def _tc_prep(x, cw, w1a, b1, blk=2000):
    return pl.pallas_call(
        _prep_body,
        grid=(N1 // blk,),
        in_specs=[
            pl.BlockSpec((blk, D), lambda i: (i, 0)),
            pl.BlockSpec((blk, D), lambda i: (i, 0)),
            pl.BlockSpec((D, D), lambda i: (0, 0)),
            pl.BlockSpec((1, D), lambda i: (0, 0)),
        ],
        out_specs=[
            pl.BlockSpec((blk, D2), lambda i: (i, 0)),
            pl.BlockSpec((blk, D2), lambda i: (i, 0)),
        ],
        out_shape=[
            jax.ShapeDtypeStruct((N1, D2), jnp.float32),
            jax.ShapeDtypeStruct((N1, D2), jnp.float32),
        ],
    )(x, cw, w1a, b1)


def _d7_body(x_ref, w_ref, c_ref, o_ref):
    d = jnp.dot(x_ref[...], w_ref[...], preferred_element_type=jnp.float32)
    o_ref[...] = c_ref[...] + jnp.concatenate([jnp.zeros_like(d), d], axis=1)


def _tc_d7(x, w, c128, blk=2000):
    return pl.pallas_call(
        _d7_body,
        grid=(N1 // blk,),
        in_specs=[
            pl.BlockSpec((blk, D), lambda i: (i, 0)),
            pl.BlockSpec((D, D), lambda i: (0, 0)),
            pl.BlockSpec((blk, D2), lambda i: (i, 0)),
        ],
        out_specs=pl.BlockSpec((blk, D2), lambda i: (i, 0)),
        out_shape=jax.ShapeDtypeStruct((N1, D2), jnp.float32),
    )(x, w, c128)


def _pf_body(pre_ref, w2_ref, b2_ref, o_ref):
    h = jnp.maximum(pre_ref[:, 0:D], 0.0)
    pf = jnp.maximum(
        jnp.dot(h, w2_ref[...], preferred_element_type=jnp.float32)
        + b2_ref[...], 0.0)
    o_ref[...] = jnp.concatenate([pf, jnp.zeros_like(pf)], axis=1)


def _tc_point_mlp(pre128, w2, b2, blk=2000):
    return pl.pallas_call(
        _pf_body,
        grid=(N_PTS // blk,),
        in_specs=[
            pl.BlockSpec((blk, D2), lambda i: (i, 0)),
            pl.BlockSpec((D, D), lambda i: (0, 0)),
            pl.BlockSpec((1, D), lambda i: (0, 0)),
        ],
        out_specs=pl.BlockSpec((blk, D2), lambda i: (i, 0)),
        out_shape=jax.ShapeDtypeStruct((N_PTS, D2), jnp.float32),
    )(pre128, w2, b2)


def _edge_act_body(pre_ref, w2_ref, b2_ref, o_ref):
    pre = pre_ref[...]
    h = jnp.maximum(pre[:, 0:D] + pre[:, D:D2], 0.0)
    ef = lax.dot_general(w2_ref[...], h, (((0,), (1,)), ((), ())),
                         preferred_element_type=jnp.float32)
    ef = jnp.maximum(ef + b2_ref[...], 0.0)
    a = lax.bitcast_convert_type(ef[0:32, :].astype(jnp.bfloat16),
                                 jnp.uint16).astype(jnp.int32)
    b = lax.bitcast_convert_type(ef[32:64, :].astype(jnp.bfloat16),
                                 jnp.uint16).astype(jnp.int32)
    packed = lax.bitwise_or(lax.shift_left(a, 16), b)
    o_ref[0] = packed.reshape(32, _MCH // 128, 128)


def _tc_edge_act(pre128, w2, b2col):
    return pl.pallas_call(
        _edge_act_body,
        grid=(_MSTEPS,),
        in_specs=[
            pl.BlockSpec((_MCH, D2), lambda i: (i, 0)),
            pl.BlockSpec((D, D), lambda i: (0, 0)),
            pl.BlockSpec((D, 1), lambda i: (0, 0)),
        ],
        out_specs=pl.BlockSpec((1, 32, _MCH // 128, 128),
                               lambda i: (i, 0, 0, 0)),
        out_shape=jax.ShapeDtypeStruct((_MSTEPS, 32, _MCH // 128, 128),
                                       jnp.int32),
    )(pre128, w2, b2col)


def _out_mlp_body(aggt_ref, w1_ref, b1_ref, w2_ref, b2_ref, o_ref):
    aggt = aggt_ref[...].reshape(D, -1)
    h = jnp.maximum(
        lax.dot_general(w1_ref[...], aggt, (((0,), (0,)), ((), ())),
                        preferred_element_type=jnp.float32) + b1_ref[...],
        0.0)
    o_ref[...] = jnp.maximum(
        lax.dot_general(h, w2_ref[...], (((0,), (0,)), ((), ())),
                        preferred_element_type=jnp.float32) + b2_ref[...],
        0.0)


def _out_mlp_res_body(aggt_ref, w1_ref, b1_ref, w2_ref, b2_ref, res_ref,
                      o_ref):
    aggt = aggt_ref[...].reshape(D, -1)
    h = jnp.maximum(
        lax.dot_general(w1_ref[...], aggt, (((0,), (0,)), ((), ())),
                        preferred_element_type=jnp.float32) + b1_ref[...],
        0.0)
    o_ref[...] = jnp.maximum(
        lax.dot_general(h, w2_ref[...], (((0,), (0,)), ((), ())),
                        preferred_element_type=jnp.float32) + b2_ref[...],
        0.0) + res_ref[...]


def _tc_out_mlp(agg4, w1, b1col, w2, b2row, res=None, blk=2048):
    body = _out_mlp_body if res is None else _out_mlp_res_body
    in_specs = [
        pl.BlockSpec((D, blk // 128, 128), lambda i: (0, i, 0)),
        pl.BlockSpec((D, D), lambda i: (0, 0)),
        pl.BlockSpec((D, 1), lambda i: (0, 0)),
        pl.BlockSpec((D, D), lambda i: (0, 0)),
        pl.BlockSpec((1, D), lambda i: (0, 0)),
    ]
    args = [agg4, w1, b1col, w2, b2row]
    if res is not None:
        in_specs.append(pl.BlockSpec((blk, D), lambda i: (i, 0)))
        args.append(res)
    return pl.pallas_call(
        body,
        grid=(_N1P // blk,),
        in_specs=in_specs,
        out_specs=pl.BlockSpec((blk, D), lambda i: (i, 0)),
        out_shape=jax.ShapeDtypeStruct((_N1P, D), jnp.float32),
    )(*args)


def _final_body(pre_ref, w2_ref, b2_ref, cw_ref, cb_ref, o_ref):
    h = jnp.maximum(pre_ref[:, D:D2], 0.0)
    h2 = jnp.maximum(
        jnp.dot(h, w2_ref[...], preferred_element_type=jnp.float32)
        + b2_ref[...], 0.0)
    o_ref[...] = (
        jnp.dot(h2, cw_ref[...], preferred_element_type=jnp.float32)
        + cb_ref[...])


def _tc_final(pre128, w2, b2, cls_w, cls_b_row, blk=2000):
    return pl.pallas_call(
        _final_body,
        grid=(N_PTS // blk,),
        in_specs=[
            pl.BlockSpec((blk, D2), lambda i: (i, 0)),
            pl.BlockSpec((D, D), lambda i: (0, 0)),
            pl.BlockSpec((1, D), lambda i: (0, 0)),
            pl.BlockSpec((D, N_CLASSES), lambda i: (0, 0)),
            pl.BlockSpec((1, N_CLASSES), lambda i: (0, 0)),
        ],
        out_specs=pl.BlockSpec((blk, N_CLASSES), lambda i: (i, 0)),
        out_shape=jax.ShapeDtypeStruct((N_PTS, N_CLASSES), jnp.float32),
    )(pre128, w2, b2, cls_w, cls_b_row)


def _sc_gather_combine(q128, table128, idx):
    nchunks = N_PTS // _GCH

    @functools.partial(
        pl.kernel,
        out_type=jax.ShapeDtypeStruct((N_PTS, D2), jnp.float32),
        mesh=_SC_MESH(),
        compiler_params=_SC_PARAMS,
        scratch_types=[
            pltpu.VMEM((_GCH,), jnp.int32),
            pltpu.VMEM((_GCH, D2), jnp.float32),
        ],
    )
    def k(q_hbm, t_hbm, idx_hbm, out_hbm, idx_v, rows_v):
        wid = lax.axis_index("s") * NCORE + lax.axis_index("c")

        def body(j, carry):
            g = j * NW + wid

            @pl.when(g < nchunks)
            def _():
                pltpu.sync_copy(idx_hbm.at[pl.ds(g * _GCH, _GCH)], idx_v)
                pltpu.sync_copy(q_hbm.at[pl.ds(g * _GCH, _GCH)], rows_v)
                pltpu.sync_copy(t_hbm.at[idx_v], rows_v, add=True)
                pltpu.sync_copy(rows_v, out_hbm.at[pl.ds(g * _GCH, _GCH)])

            return carry

        lax.fori_loop(0, (nchunks + NW - 1) // NW, body, 0)

    return k(q128, table128, idx)


def _sc_two_gather(a128, m128, src, dst):
    gch = 400
    steps = E_PAD // gch // NW

    @functools.partial(
        pl.kernel,
        out_type=jax.ShapeDtypeStruct((E_PAD, D2), jnp.float32),
        mesh=_SC_MESH(),
        scratch_types=[
            pltpu.VMEM((gch,), jnp.int32),
            pltpu.VMEM((gch,), jnp.int32),
            pltpu.VMEM((gch,), jnp.int32),
            pltpu.VMEM((gch,), jnp.int32),
            pltpu.VMEM((gch, D2), jnp.float32),
            pltpu.VMEM((gch, D2), jnp.float32),
            pltpu.SemaphoreType.DMA,
            pltpu.SemaphoreType.DMA,
            pltpu.SemaphoreType.DMA,
            pltpu.SemaphoreType.DMA,
        ],
    )
    def k(a_hbm, m_hbm, src_hbm, dst_hbm, out_hbm, is0, is1, id0, id1,
          rows0, rows1, sa0, sa1, so0, so1):
        wid = lax.axis_index("s") * NCORE + lax.axis_index("c")

        def idx_in(j, isv, idv):
            g = j * NW + wid
            pltpu.sync_copy(src_hbm.at[pl.ds(g * gch, gch)], isv)
            pltpu.sync_copy(dst_hbm.at[pl.ds(g * gch, gch)], idv)

        def gather_start(isv, rows, sa):
            pltpu.async_copy(a_hbm.at[isv], rows, sa)

        def gather_wait(isv, rows, sa):
            pltpu.make_async_copy(a_hbm.at[isv], rows, sa).wait()

        def out_start(j, rows, so):
            g = j * NW + wid
            pltpu.async_copy(rows, out_hbm.at[pl.ds(g * gch, gch)], so)

        def out_wait(j, rows, so):
            g = j * NW + wid
            pltpu.make_async_copy(
                rows, out_hbm.at[pl.ds(g * gch, gch)], so).wait()

        idx_in(0, is0, id0)
        gather_start(is0, rows0, sa0)
        idx_in(1, is1, id1)
        gather_start(is1, rows1, sa1)

        def body(j, carry):
            b = lax.rem(j, 2)

            def finish(isv, idv, rows, sa, so):
                gather_wait(isv, rows, sa)
                pltpu.sync_copy(m_hbm.at[idv], rows, add=True)
                out_start(j, rows, so)

                @pl.when(j + 2 < steps)
                def _():
                    out_wait(j, rows, so)
                    idx_in(j + 2, isv, idv)
                    gather_start(isv, rows, sa)

            @pl.when(b == 0)
            def _():
                finish(is0, id0, rows0, sa0, so0)

            @pl.when(b == 1)
            def _():
                finish(is1, id1, rows1, sa1, so1)

            return carry

        lax.fori_loop(0, steps, body, 0)
        out_wait(steps - 2, rows0, so0)
        out_wait(steps - 1, rows1, so1)

    return k(a128, m128, src, dst)


def _sc_scatter_add(pf128, labels):
    nchunks = N_PTS // _KCH
    tile_rows = _NQP // NSUB

    @functools.partial(
        pl.kernel,
        out_type=jax.ShapeDtypeStruct((8, _NQP, D2), jnp.float32),
        mesh=_SC_MESH(),
        compiler_params=_SC_PARAMS,
        scratch_types=[
            pltpu.VMEM((_KCH,), jnp.int32),
            pltpu.VMEM((_KCH,), jnp.int32),
            pltpu.VMEM((_KCH, D2), jnp.float32),
            pltpu.VMEM((56, D2), jnp.float32),
            pltpu.VMEM_SHARED((_NQP, D2), jnp.float32),
        ],
    )
    def k(pf_hbm, lab_hbm, out_hbm, idx_v, fidx_v, rows_v, zbuf, acc):
        c = lax.axis_index("c")
        sid = lax.axis_index("s")

        def zrow(i, carry):
            def zlane(r, carry2):
                zbuf[i, pl.ds(r * 16, 16)] = jnp.zeros((16,), jnp.float32)
                return carry2

            return lax.fori_loop(0, 8, zlane, carry)

        lax.fori_loop(0, 56, zrow, 0)

        def octant(qi, carry):
            q = c * 4 + qi
            qbase = q * _NQ

            def zcp(j, carry2):
                pltpu.sync_copy(
                    zbuf, acc.at[pl.ds(sid * tile_rows + j * 56, 56)])
                return carry2

            lax.fori_loop(0, tile_rows // 56, zcp, 0)
            plsc.subcore_barrier()

            def body(j, carry2):
                g = j * NSUB + sid

                @pl.when(g < nchunks)
                def _():
                    pltpu.sync_copy(lab_hbm.at[pl.ds(g * _KCH, _KCH)],
                                    idx_v)

                    def remap(t, carry3):
                        iv = idx_v[pl.ds(t * 16, 16)]
                        lidx = iv - qbase
                        inq = (iv >= qbase) & (lidx < _NQ)
                        dummy = jnp.full((16,), _NQ, jnp.int32) + (t & 31)
                        fidx_v[pl.ds(t * 16, 16)] = jnp.where(
                            inq, lidx, dummy)
                        return carry3

                    lax.fori_loop(0, _KCH // 16, remap, 0)
                    pltpu.sync_copy(pf_hbm.at[pl.ds(g * _KCH, _KCH)],
                                    rows_v)
                    pltpu.sync_copy(rows_v, acc.at[fidx_v], add=True)

                return carry2

            lax.fori_loop(0, (nchunks + NSUB - 1) // NSUB, body, 0)
            plsc.subcore_barrier()
            pltpu.sync_copy(
                acc.at[pl.ds(sid * tile_rows, tile_rows)],
                out_hbm.at[q, pl.ds(sid * tile_rows, tile_rows)])
            plsc.subcore_barrier()
            return carry

        lax.fori_loop(0, 4, octant, 0)

    return k(pf128, labels)


def _sc_scatter_max(ef4, dst):
    nrow = _MCH // 128
    ngrp = _MCH // 16
    himask = jnp.int32(-65536)

    @functools.partial(
        pl.kernel,
        out_type=jax.ShapeDtypeStruct((D, _N1P // 128, 128), jnp.float32),
        mesh=_SC_MESH(),
        compiler_params=_SC_PARAMS,
        scratch_types=[
            pltpu.VMEM((_MCH,), jnp.int32),
            pltpu.VMEM((nrow, 128), jnp.int32),
            pltpu.VMEM((_N1P,), jnp.int32),
            pltpu.VMEM((8, 128), jnp.float32),
        ],
    )
    def k(ef_hbm, dst_hbm, out_hbm, idx_v, pv_v, acc, stage):
        wid = lax.axis_index("s") * NCORE + lax.axis_index("c")

        def zrow(i, carry):
            acc[pl.ds(i * 16, 16)] = jnp.zeros((16,), jnp.int32)
            return carry

        lax.fori_loop(0, _N1P // 16, zrow, 0)

        def step(j, carry):
            pltpu.sync_copy(dst_hbm.at[pl.ds(j * _MCH, _MCH)], idx_v)
            pltpu.sync_copy(ef_hbm.at[j, wid], pv_v)

            def grp(g4, confl):
                for u in range(4):
                    g = g4 * 4 + u
                    iv = idx_v[pl.ds(g * 16, 16)]
                    r = lax.shift_right_logical(g, 3)
                    o = lax.bitwise_and(g, 7) * 16
                    pv = pv_v[r, pl.ds(o, 16)]
                    vhi = lax.shift_right_logical(pv, 16)
                    vlo = lax.bitwise_and(pv, 65535)
                    cur = plsc.load_gather(acc, (iv,))
                    mhi = jnp.maximum(lax.shift_right_logical(cur, 16),
                                      vhi)
                    mlo = jnp.maximum(lax.bitwise_and(cur, 65535), vlo)
                    plsc.store_scatter(
                        acc, (iv,),
                        lax.bitwise_or(lax.shift_left(mhi, 16), mlo))
                    rb = plsc.load_gather(acc, (iv,))
                    confl = (confl
                             | (vhi > lax.shift_right_logical(rb, 16))
                             | (vlo > lax.bitwise_and(rb, 65535)))
                return confl

            confl = lax.fori_loop(0, ngrp // 4, grp,
                                  jnp.zeros((16,), jnp.bool_))
            cnt = jnp.max(plsc.all_reduce_population_count(confl))

            def fix_round(c):
                def grp2(g, confl2):
                    iv = idx_v[pl.ds(g * 16, 16)]
                    r = lax.shift_right_logical(g, 3)
                    o = lax.bitwise_and(g, 7) * 16
                    pv = pv_v[r, pl.ds(o, 16)]
                    vhi = lax.shift_right_logical(pv, 16)
                    vlo = lax.bitwise_and(pv, 65535)
                    cur = plsc.load_gather(acc, (iv,))
                    chi = lax.shift_right_logical(cur, 16)
                    clo = lax.bitwise_and(cur, 65535)
                    need = (vhi > chi) | (vlo > clo)
                    merged = lax.bitwise_or(
                        lax.shift_left(jnp.maximum(chi, vhi), 16),
                        jnp.maximum(clo, vlo))
                    plsc.store_scatter(acc, (iv,), merged, mask=need)
                    rb = plsc.load_gather(acc, (iv,))
                    return (confl2
                            | (vhi > lax.shift_right_logical(rb, 16))
                            | (vlo > lax.bitwise_and(rb, 65535)))

                c2 = lax.fori_loop(0, ngrp, grp2,
                                   jnp.zeros((16,), jnp.bool_))
                return jnp.max(plsc.all_reduce_population_count(c2))

            lax.while_loop(lambda c: c > 0, fix_round, cnt)
            return carry

        lax.fori_loop(0, _MSTEPS, step, 0)

        def wcol(col, hi_half):
            def wchunk(j, carry):
                def wrow(t, carry2):
                    r = lax.shift_right_logical(t, 3)
                    o = lax.bitwise_and(t, 7) * 16
                    v = acc[pl.ds(j * 1024 + r * 128 + o, 16)]
                    bits = lax.bitwise_and(v, himask) if hi_half else \
                        lax.shift_left(v, 16)
                    stage[r, pl.ds(o, 16)] = plsc.bitcast(bits, jnp.float32)
                    return carry2

                lax.fori_loop(0, 64, wrow, 0)
                pltpu.sync_copy(stage, out_hbm.at[col, pl.ds(j * 8, 8)])
                return carry

            lax.fori_loop(0, _N1P // 1024, wchunk, 0)

        wcol(wid, True)
        wcol(wid + 32, False)

    return k(ef4, dst)


def kernel(remission, points, l1_cluster_centers, l2_cluster_centers,
           l1_edges, l2_edges, l1_labels, l2_labels,
           l1_ffn, l2_edge_mlp, l2_out_mlp, l6_edge_mlp, l6_out_mlp,
           l7_fbn, cls_W, cls_b):
    del l2_cluster_centers, l2_edges, l2_labels
    f32 = jnp.float32
    centers = l1_cluster_centers.astype(f32)
    labels = l1_labels.astype(jnp.int32)
    src = l1_edges[:, 0].astype(jnp.int32)
    dst = l1_edges[:, 1].astype(jnp.int32)
    pad_idx = (jnp.arange(E_PAD - E1, dtype=jnp.int32) * 97) % N1
    src_g = jnp.concatenate([src, pad_idx])
    dst_g = jnp.concatenate([dst, pad_idx])
    dst_s = jnp.pad(dst, (0, E_PAD - E1), constant_values=N1)

    w1_1, b1_1, w2_1, b2_1 = l1_ffn
    w1_2, b1_2, w2_2, b2_2 = l2_edge_mlp
    w1_6, b1_6, w2_6, b2_6 = l6_edge_mlp
    w1_7, b1_7, w2_7, b2_7 = l7_fbn
    w1p = w1_1[1:4]
    w1r = w1_1[0:1]
    w2b = w1_2[D:D + 3]
    w6b = w1_6[D:D + 3]
    w7a = w1_7[0:D]
    w7b = w1_7[D:D + 3]

    z64 = jnp.zeros((3, D), f32)
    wc = jnp.concatenate([-w1p, z64, w2b, w6b, z64, -w7b], axis=1)
    wc4 = jnp.pad(wc, ((0, 1), (0, 0)))
    centers4 = jnp.pad(centers, ((0, 0), (0, 1)))
    cw_all = _tc_matmul_bias(centers4, wc4, jnp.zeros((1, 6 * D), f32),
                             blk=2000)
    tab1 = cw_all[:, 0:D2]
    cw2 = cw_all[:, D2:D2 + D]
    cw6 = cw_all[:, D2 + D:D2 + 2 * D]
    tab7b = cw_all[:, D2 + 2 * D:D2 + 2 * D + D2]

    pts4 = jnp.concatenate([remission.astype(f32), points.astype(f32)],
                           axis=1)
    w4 = jnp.concatenate([
        jnp.concatenate([w1r, w1p], axis=0),
        jnp.concatenate([jnp.zeros((1, D), f32), w7b], axis=0),
    ], axis=1)
    bq = jnp.concatenate([b1_1, b1_7])[None, :]
    q_cat = _tc_matmul_bias(pts4, w4, bq, blk=2000)

    pre1 = _sc_gather_combine(q_cat, tab1, labels)
    pf128 = _tc_point_mlp(pre1, w2_1, b2_1[None, :])
    t18 = _sc_scatter_add(pf128, labels)
    t1 = t18[:, :_NQ, :D].reshape(N1, D)

    a2, m2 = _tc_prep(t1, cw2, w1_2[0:D], b1_2[None, :])
    pre2 = _sc_two_gather(a2, m2, src_g, dst_g)
    eft2 = _tc_edge_act(pre2, w2_2, b2_2[:, None])
    agg2 = _sc_scatter_max(eft2, dst_s)
    w1o2, b1o2, w2o2, b2o2 = l2_out_mlp
    t2p = _tc_out_mlp(agg2, w1o2, b1o2[:, None], w2o2, b2o2[None, :])
    t2 = t2p[:N1]

    a6, m6 = _tc_prep(t2, cw6, w1_6[0:D], b1_6[None, :])
    pre6 = _sc_two_gather(a6, m6, src_g, dst_g)
    eft6 = _tc_edge_act(pre6, w2_6, b2_6[:, None])
    agg6 = _sc_scatter_max(eft6, dst_s)
    w1o6, b1o6, w2o6, b2o6 = l6_out_mlp
    t6p = _tc_out_mlp(agg6, w1o6, b1o6[:, None], w2o6, b2o6[None, :],
                      res=t2p)
    t6 = t6p[:N1]

    d7 = _tc_d7(t6, w7a, tab7b)
    pre7 = _sc_gather_combine(q_cat, d7, labels)
    return _tc_final(pre7, w2_7, b2_7[None, :], cls_W.astype(f32),
                     cls_b.astype(f32)[None, :])


# speedup vs baseline: 1.8360x; 1.1167x over previous
import functools

import jax
import jax.numpy as jnp
from jax import lax
from jax.experimental import pallas as pl
from jax.experimental.pallas import tpu as pltpu
from jax.experimental.pallas import tpu_sc as plsc

N_PTS = 100000
N1 = 50000
E1 = 800000
E_PAD = 819200
D = 64
D2 = 128
N_CLASSES = 20

NCORE = 2
NSUB = 16
NW = NCORE * NSUB

_GCH = 800
_KCH = 400
_NQ = 6250
_NQP = 6272
_MCH = 4096
_MSTEPS = E_PAD // _MCH
_N1P = 51200

_SC_MESH = functools.partial(
    plsc.VectorSubcoreMesh, core_axis_name="c", subcore_axis_name="s")
_SC_PARAMS = pltpu.CompilerParams(needs_layout_passes=False)


def _mm_bias_body(x_ref, w_ref, b_ref, o_ref):
    o_ref[...] = (
        jnp.dot(x_ref[...], w_ref[...], preferred_element_type=jnp.float32)
        + b_ref[...])


def _tc_matmul_bias(x, w, b, blk):
    n, k = x.shape
    f = w.shape[1]
    return pl.pallas_call(
        _mm_bias_body,
        grid=(n // blk,),
        in_specs=[
            pl.BlockSpec((blk, k), lambda i: (i, 0)),
            pl.BlockSpec((k, f), lambda i: (0, 0)),
            pl.BlockSpec((1, f), lambda i: (0, 0)),
        ],
        out_specs=pl.BlockSpec((blk, f), lambda i: (i, 0)),
        out_shape=jax.ShapeDtypeStruct((n, f), jnp.float32),
    )(x, w, b)


def _prep_body(x_ref, cw_ref, w_ref, b_ref, a_ref, m_ref):
    cw = cw_ref[...]
    z = jnp.zeros_like(cw)
    a = (jnp.dot(x_ref[...], w_ref[...], preferred_element_type=jnp.float32)
         + cw + b_ref[...])
    a_ref[...] = jnp.concatenate([a, z], axis=1)
    m_ref[...] = jnp.concatenate([z, -cw], axis=1)


def _tc_prep(x, cw, w1a, b1, blk=2000):
    return pl.pallas_call(
        _prep_body,
        grid=(N1 // blk,),
        in_specs=[
            pl.BlockSpec((blk, D), lambda i: (i, 0)),
            pl.BlockSpec((blk, D), lambda i: (i, 0)),
            pl.BlockSpec((D, D), lambda i: (0, 0)),
            pl.BlockSpec((1, D), lambda i: (0, 0)),
        ],
        out_specs=[
            pl.BlockSpec((blk, D2), lambda i: (i, 0)),
            pl.BlockSpec((blk, D2), lambda i: (i, 0)),
        ],
        out_shape=[
            jax.ShapeDtypeStruct((N1, D2), jnp.float32),
            jax.ShapeDtypeStruct((N1, D2), jnp.float32),
        ],
    )(x, cw, w1a, b1)


def _d7_body(x_ref, w_ref, c_ref, o_ref):
    d = jnp.dot(x_ref[...], w_ref[...], preferred_element_type=jnp.float32)
    o_ref[...] = c_ref[...] + jnp.concatenate([jnp.zeros_like(d), d], axis=1)


def _tc_d7(x, w, c128, blk=2000):
    return pl.pallas_call(
        _d7_body,
        grid=(N1 // blk,),
        in_specs=[
            pl.BlockSpec((blk, D), lambda i: (i, 0)),
            pl.BlockSpec((D, D), lambda i: (0, 0)),
            pl.BlockSpec((blk, D2), lambda i: (i, 0)),
        ],
        out_specs=pl.BlockSpec((blk, D2), lambda i: (i, 0)),
        out_shape=jax.ShapeDtypeStruct((N1, D2), jnp.float32),
    )(x, w, c128)


def _pf_body(pre_ref, w2_ref, b2_ref, o_ref):
    h = jnp.maximum(pre_ref[:, 0:D], 0.0)
    pf = jnp.maximum(
        jnp.dot(h, w2_ref[...], preferred_element_type=jnp.float32)
        + b2_ref[...], 0.0)
    o_ref[...] = jnp.concatenate([pf, jnp.zeros_like(pf)], axis=1)


def _tc_point_mlp(pre128, w2, b2, blk=2000):
    return pl.pallas_call(
        _pf_body,
        grid=(N_PTS // blk,),
        in_specs=[
            pl.BlockSpec((blk, D2), lambda i: (i, 0)),
            pl.BlockSpec((D, D), lambda i: (0, 0)),
            pl.BlockSpec((1, D), lambda i: (0, 0)),
        ],
        out_specs=pl.BlockSpec((blk, D2), lambda i: (i, 0)),
        out_shape=jax.ShapeDtypeStruct((N_PTS, D2), jnp.float32),
    )(pre128, w2, b2)


def _edge_act_body(pre_ref, w2_ref, b2_ref, o_ref):
    pre = pre_ref[...]
    h = jnp.maximum(pre[:, 0:D] + pre[:, D:D2], 0.0)
    ef = lax.dot_general(w2_ref[...], h, (((0,), (1,)), ((), ())),
                         preferred_element_type=jnp.float32)
    ef = jnp.maximum(ef + b2_ref[...], 0.0)
    a = lax.bitcast_convert_type(ef[0:32, :].astype(jnp.bfloat16),
                                 jnp.uint16).astype(jnp.int32)
    b = lax.bitcast_convert_type(ef[32:64, :].astype(jnp.bfloat16),
                                 jnp.uint16).astype(jnp.int32)
    packed = lax.bitwise_or(lax.shift_left(a, 16), b)
    o_ref[0] = packed.reshape(32, _MCH // 128, 128)


def _tc_edge_act(pre128, w2, b2col):
    return pl.pallas_call(
        _edge_act_body,
        grid=(_MSTEPS,),
        in_specs=[
            pl.BlockSpec((_MCH, D2), lambda i: (i, 0)),
            pl.BlockSpec((D, D), lambda i: (0, 0)),
            pl.BlockSpec((D, 1), lambda i: (0, 0)),
        ],
        out_specs=pl.BlockSpec((1, 32, _MCH // 128, 128),
                               lambda i: (i, 0, 0, 0)),
        out_shape=jax.ShapeDtypeStruct((_MSTEPS, 32, _MCH // 128, 128),
                                       jnp.int32),
    )(pre128, w2, b2col)


def _out_mlp_body(aggt_ref, w1_ref, b1_ref, w2_ref, b2_ref, o_ref):
    aggt = aggt_ref[...].reshape(D, -1)
    h = jnp.maximum(
        lax.dot_general(w1_ref[...], aggt, (((0,), (0,)), ((), ())),
                        preferred_element_type=jnp.float32) + b1_ref[...],
        0.0)
    o_ref[...] = jnp.maximum(
        lax.dot_general(h, w2_ref[...], (((0,), (0,)), ((), ())),
                        preferred_element_type=jnp.float32) + b2_ref[...],
        0.0)


def _out_mlp_res_body(aggt_ref, w1_ref, b1_ref, w2_ref, b2_ref, res_ref,
                      o_ref):
    aggt = aggt_ref[...].reshape(D, -1)
    h = jnp.maximum(
        lax.dot_general(w1_ref[...], aggt, (((0,), (0,)), ((), ())),
                        preferred_element_type=jnp.float32) + b1_ref[...],
        0.0)
    o_ref[...] = jnp.maximum(
        lax.dot_general(h, w2_ref[...], (((0,), (0,)), ((), ())),
                        preferred_element_type=jnp.float32) + b2_ref[...],
        0.0) + res_ref[...]


def _tc_out_mlp(agg4, w1, b1col, w2, b2row, res=None, blk=2048):
    body = _out_mlp_body if res is None else _out_mlp_res_body
    in_specs = [
        pl.BlockSpec((D, blk // 128, 128), lambda i: (0, i, 0)),
        pl.BlockSpec((D, D), lambda i: (0, 0)),
        pl.BlockSpec((D, 1), lambda i: (0, 0)),
        pl.BlockSpec((D, D), lambda i: (0, 0)),
        pl.BlockSpec((1, D), lambda i: (0, 0)),
    ]
    args = [agg4, w1, b1col, w2, b2row]
    if res is not None:
        in_specs.append(pl.BlockSpec((blk, D), lambda i: (i, 0)))
        args.append(res)
    return pl.pallas_call(
        body,
        grid=(_N1P // blk,),
        in_specs=in_specs,
        out_specs=pl.BlockSpec((blk, D), lambda i: (i, 0)),
        out_shape=jax.ShapeDtypeStruct((_N1P, D), jnp.float32),
    )(*args)


def _final_body(pre_ref, w2_ref, b2_ref, cw_ref, cb_ref, o_ref):
    h = jnp.maximum(pre_ref[:, D:D2], 0.0)
    h2 = jnp.maximum(
        jnp.dot(h, w2_ref[...], preferred_element_type=jnp.float32)
        + b2_ref[...], 0.0)
    o_ref[...] = (
        jnp.dot(h2, cw_ref[...], preferred_element_type=jnp.float32)
        + cb_ref[...])


def _tc_final(pre128, w2, b2, cls_w, cls_b_row, blk=2000):
    return pl.pallas_call(
        _final_body,
        grid=(N_PTS // blk,),
        in_specs=[
            pl.BlockSpec((blk, D2), lambda i: (i, 0)),
            pl.BlockSpec((D, D), lambda i: (0, 0)),
            pl.BlockSpec((1, D), lambda i: (0, 0)),
            pl.BlockSpec((D, N_CLASSES), lambda i: (0, 0)),
            pl.BlockSpec((1, N_CLASSES), lambda i: (0, 0)),
        ],
        out_specs=pl.BlockSpec((blk, N_CLASSES), lambda i: (i, 0)),
        out_shape=jax.ShapeDtypeStruct((N_PTS, N_CLASSES), jnp.float32),
    )(pre128, w2, b2, cls_w, cls_b_row)


def _sc_gather_combine(q128, table128, idx):
    nchunks = N_PTS // _GCH

    @functools.partial(
        pl.kernel,
        out_type=jax.ShapeDtypeStruct((N_PTS, D2), jnp.float32),
        mesh=_SC_MESH(),
        compiler_params=_SC_PARAMS,
        scratch_types=[
            pltpu.VMEM((_GCH,), jnp.int32),
            pltpu.VMEM((_GCH, D2), jnp.float32),
        ],
    )
    def k(q_hbm, t_hbm, idx_hbm, out_hbm, idx_v, rows_v):
        wid = lax.axis_index("s") * NCORE + lax.axis_index("c")

        def body(j, carry):
            g = j * NW + wid

            @pl.when(g < nchunks)
            def _():
                pltpu.sync_copy(idx_hbm.at[pl.ds(g * _GCH, _GCH)], idx_v)
                pltpu.sync_copy(q_hbm.at[pl.ds(g * _GCH, _GCH)], rows_v)
                pltpu.sync_copy(t_hbm.at[idx_v], rows_v, add=True)
                pltpu.sync_copy(rows_v, out_hbm.at[pl.ds(g * _GCH, _GCH)])

            return carry

        lax.fori_loop(0, (nchunks + NW - 1) // NW, body, 0)

    return k(q128, table128, idx)


def _sc_two_gather(a128, m128, src, dst):
    gch = 400
    steps = E_PAD // gch // NW

    @functools.partial(
        pl.kernel,
        out_type=jax.ShapeDtypeStruct((E_PAD, D2), jnp.float32),
        mesh=_SC_MESH(),
        scratch_types=[
            pltpu.VMEM((gch,), jnp.int32),
            pltpu.VMEM((gch,), jnp.int32),
            pltpu.VMEM((gch,), jnp.int32),
            pltpu.VMEM((gch,), jnp.int32),
            pltpu.VMEM((gch, D2), jnp.float32),
            pltpu.VMEM((gch, D2), jnp.float32),
            pltpu.SemaphoreType.DMA,
            pltpu.SemaphoreType.DMA,
            pltpu.SemaphoreType.DMA,
            pltpu.SemaphoreType.DMA,
        ],
    )
    def k(a_hbm, m_hbm, src_hbm, dst_hbm, out_hbm, is0, is1, id0, id1,
          rows0, rows1, sa0, sa1, so0, so1):
        wid = lax.axis_index("s") * NCORE + lax.axis_index("c")

        def idx_in(j, isv, idv):
            g = j * NW + wid
            pltpu.sync_copy(src_hbm.at[pl.ds(g * gch, gch)], isv)
            pltpu.sync_copy(dst_hbm.at[pl.ds(g * gch, gch)], idv)

        def gather_start(isv, rows, sa):
            pltpu.async_copy(a_hbm.at[isv], rows, sa)

        def gather_wait(isv, rows, sa):
            pltpu.make_async_copy(a_hbm.at[isv], rows, sa).wait()

        def out_start(j, rows, so):
            g = j * NW + wid
            pltpu.async_copy(rows, out_hbm.at[pl.ds(g * gch, gch)], so)

        def out_wait(j, rows, so):
            g = j * NW + wid
            pltpu.make_async_copy(
                rows, out_hbm.at[pl.ds(g * gch, gch)], so).wait()

        idx_in(0, is0, id0)
        gather_start(is0, rows0, sa0)
        idx_in(1, is1, id1)
        gather_start(is1, rows1, sa1)

        def body(j, carry):
            b = lax.rem(j, 2)

            def finish(isv, idv, rows, sa, so):
                gather_wait(isv, rows, sa)
                pltpu.sync_copy(m_hbm.at[idv], rows, add=True)
                out_start(j, rows, so)

                @pl.when(j + 2 < steps)
                def _():
                    out_wait(j, rows, so)
                    idx_in(j + 2, isv, idv)
                    gather_start(isv, rows, sa)

            @pl.when(b == 0)
            def _():
                finish(is0, id0, rows0, sa0, so0)

            @pl.when(b == 1)
            def _():
                finish(is1, id1, rows1, sa1, so1)

            return carry

        lax.fori_loop(0, steps, body, 0)
        out_wait(steps - 2, rows0, so0)
        out_wait(steps - 1, rows1, so1)

    return k(a128, m128, src, dst)


def _sc_scatter_add(pf128, labels):
    nchunks = N_PTS // _KCH
    tile_rows = _NQP // NSUB

    @functools.partial(
        pl.kernel,
        out_type=jax.ShapeDtypeStruct((8, _NQP, D2), jnp.float32),
        mesh=_SC_MESH(),
        compiler_params=_SC_PARAMS,
        scratch_types=[
            pltpu.VMEM((_KCH,), jnp.int32),
            pltpu.VMEM((_KCH,), jnp.int32),
            pltpu.VMEM((_KCH, D2), jnp.float32),
            pltpu.VMEM((56, D2), jnp.float32),
            pltpu.VMEM_SHARED((_NQP, D2), jnp.float32),
        ],
    )
    def k(pf_hbm, lab_hbm, out_hbm, idx_v, fidx_v, rows_v, zbuf, acc):
        c = lax.axis_index("c")
        sid = lax.axis_index("s")

        def zrow(i, carry):
            def zlane(r, carry2):
                zbuf[i, pl.ds(r * 16, 16)] = jnp.zeros((16,), jnp.float32)
                return carry2

            return lax.fori_loop(0, 8, zlane, carry)

        lax.fori_loop(0, 56, zrow, 0)

        def octant(qi, carry):
            q = c * 4 + qi
            qbase = q * _NQ

            def zcp(j, carry2):
                pltpu.sync_copy(
                    zbuf, acc.at[pl.ds(sid * tile_rows + j * 56, 56)])
                return carry2

            lax.fori_loop(0, tile_rows // 56, zcp, 0)
            plsc.subcore_barrier()

            def body(j, carry2):
                g = j * NSUB + sid

                @pl.when(g < nchunks)
                def _():
                    pltpu.sync_copy(lab_hbm.at[pl.ds(g * _KCH, _KCH)],
                                    idx_v)

                    def remap(t, carry3):
                        iv = idx_v[pl.ds(t * 16, 16)]
                        lidx = iv - qbase
                        inq = (iv >= qbase) & (lidx < _NQ)
                        dummy = jnp.full((16,), _NQ, jnp.int32) + (t & 31)
                        fidx_v[pl.ds(t * 16, 16)] = jnp.where(
                            inq, lidx, dummy)
                        return carry3

                    lax.fori_loop(0, _KCH // 16, remap, 0)
                    pltpu.sync_copy(pf_hbm.at[pl.ds(g * _KCH, _KCH)],
                                    rows_v)
                    pltpu.sync_copy(rows_v, acc.at[fidx_v], add=True)

                return carry2

            lax.fori_loop(0, (nchunks + NSUB - 1) // NSUB, body, 0)
            plsc.subcore_barrier()
            pltpu.sync_copy(
                acc.at[pl.ds(sid * tile_rows, tile_rows)],
                out_hbm.at[q, pl.ds(sid * tile_rows, tile_rows)])
            plsc.subcore_barrier()
            return carry

        lax.fori_loop(0, 4, octant, 0)

    return k(pf128, labels)


def _sc_scatter_max(ef4, dst):
    nrow = _MCH // 128
    ngrp = _MCH // 16
    himask = jnp.int32(-65536)

    @functools.partial(
        pl.kernel,
        out_type=jax.ShapeDtypeStruct((D, _N1P // 128, 128), jnp.float32),
        mesh=_SC_MESH(),
        compiler_params=_SC_PARAMS,
        scratch_types=[
            pltpu.VMEM((_MCH,), jnp.int32),
            pltpu.VMEM((_MCH,), jnp.int32),
            pltpu.VMEM((nrow, 128), jnp.int32),
            pltpu.VMEM((nrow, 128), jnp.int32),
            pltpu.VMEM((_N1P,), jnp.int32),
            pltpu.VMEM((8, 128), jnp.float32),
            pltpu.SemaphoreType.DMA,
            pltpu.SemaphoreType.DMA,
        ],
    )
    def k(ef_hbm, dst_hbm, out_hbm, idx0, idx1, pv0, pv1, acc, stage,
          sem0, sem1):
        wid = lax.axis_index("s") * NCORE + lax.axis_index("c")

        def zrow(i, carry):
            acc[pl.ds(i * 16, 16)] = jnp.zeros((16,), jnp.int32)
            return carry

        lax.fori_loop(0, _N1P // 16, zrow, 0)

        def in_start(j, idx_v, pv_v, sem):
            pltpu.async_copy(dst_hbm.at[pl.ds(j * _MCH, _MCH)], idx_v, sem)
            pltpu.async_copy(ef_hbm.at[j, wid], pv_v, sem)

        def in_wait(j, idx_v, pv_v, sem):
            pltpu.make_async_copy(
                dst_hbm.at[pl.ds(j * _MCH, _MCH)], idx_v, sem).wait()
            pltpu.make_async_copy(ef_hbm.at[j, wid], pv_v, sem).wait()

        in_start(0, idx0, pv0, sem0)
        in_start(1, idx1, pv1, sem1)

        def step(j, carry):
            b = lax.rem(j, 2)

            def run(idx_v, pv_v, sem):
                in_wait(j, idx_v, pv_v, sem)
                def grp(g4, confl):
                    for u in range(4):
                        g = g4 * 4 + u
                        iv = idx_v[pl.ds(g * 16, 16)]
                        r = lax.shift_right_logical(g, 3)
                        o = lax.bitwise_and(g, 7) * 16
                        pv = pv_v[r, pl.ds(o, 16)]
                        vhi = lax.shift_right_logical(pv, 16)
                        vlo = lax.bitwise_and(pv, 65535)
                        cur = plsc.load_gather(acc, (iv,))
                        mhi = jnp.maximum(lax.shift_right_logical(cur, 16),
                                          vhi)
                        mlo = jnp.maximum(lax.bitwise_and(cur, 65535), vlo)
                        plsc.store_scatter(
                            acc, (iv,),
                            lax.bitwise_or(lax.shift_left(mhi, 16), mlo))
                        rb = plsc.load_gather(acc, (iv,))
                        confl = (confl
                                 | (vhi > lax.shift_right_logical(rb, 16))
                                 | (vlo > lax.bitwise_and(rb, 65535)))
                    return confl

                confl = lax.fori_loop(0, ngrp // 4, grp,
                                      jnp.zeros((16,), jnp.bool_))
                cnt = jnp.max(plsc.all_reduce_population_count(confl))

                def fix_round(c):
                    def grp2(g, confl2):
                        iv = idx_v[pl.ds(g * 16, 16)]
                        r = lax.shift_right_logical(g, 3)
                        o = lax.bitwise_and(g, 7) * 16
                        pv = pv_v[r, pl.ds(o, 16)]
                        vhi = lax.shift_right_logical(pv, 16)
                        vlo = lax.bitwise_and(pv, 65535)
                        cur = plsc.load_gather(acc, (iv,))
                        chi = lax.shift_right_logical(cur, 16)
                        clo = lax.bitwise_and(cur, 65535)
                        need = (vhi > chi) | (vlo > clo)
                        merged = lax.bitwise_or(
                            lax.shift_left(jnp.maximum(chi, vhi), 16),
                            jnp.maximum(clo, vlo))
                        plsc.store_scatter(acc, (iv,), merged, mask=need)
                        rb = plsc.load_gather(acc, (iv,))
                        return (confl2
                                | (vhi > lax.shift_right_logical(rb, 16))
                                | (vlo > lax.bitwise_and(rb, 65535)))

                    c2 = lax.fori_loop(0, ngrp, grp2,
                                       jnp.zeros((16,), jnp.bool_))
                    return jnp.max(plsc.all_reduce_population_count(c2))

                lax.while_loop(lambda c: c > 0, fix_round, cnt)

                @pl.when(j + 2 < _MSTEPS)
                def _():
                    in_start(j + 2, idx_v, pv_v, sem)

            @pl.when(b == 0)
            def _():
                run(idx0, pv0, sem0)

            @pl.when(b == 1)
            def _():
                run(idx1, pv1, sem1)

            return carry

        lax.fori_loop(0, _MSTEPS, step, 0)

        def wcol(col, hi_half):
            def wchunk(j, carry):
                def wrow(t, carry2):
                    r = lax.shift_right_logical(t, 3)
                    o = lax.bitwise_and(t, 7) * 16
                    v = acc[pl.ds(j * 1024 + r * 128 + o, 16)]
                    bits = lax.bitwise_and(v, himask) if hi_half else \
                        lax.shift_left(v, 16)
                    stage[r, pl.ds(o, 16)] = plsc.bitcast(bits, jnp.float32)
                    return carry2

                lax.fori_loop(0, 64, wrow, 0)
                pltpu.sync_copy(stage, out_hbm.at[col, pl.ds(j * 8, 8)])
                return carry

            lax.fori_loop(0, _N1P // 1024, wchunk, 0)

        wcol(wid, True)
        wcol(wid + 32, False)

    return k(ef4, dst)


def kernel(remission, points, l1_cluster_centers, l2_cluster_centers,
           l1_edges, l2_edges, l1_labels, l2_labels,
           l1_ffn, l2_edge_mlp, l2_out_mlp, l6_edge_mlp, l6_out_mlp,
           l7_fbn, cls_W, cls_b):
    del l2_cluster_centers, l2_edges, l2_labels
    f32 = jnp.float32
    centers = l1_cluster_centers.astype(f32)
    labels = l1_labels.astype(jnp.int32)
    src = l1_edges[:, 0].astype(jnp.int32)
    dst = l1_edges[:, 1].astype(jnp.int32)
    pad_idx = (jnp.arange(E_PAD - E1, dtype=jnp.int32) * 97) % N1
    src_g = jnp.concatenate([src, pad_idx])
    dst_g = jnp.concatenate([dst, pad_idx])
    dst_s = jnp.pad(dst, (0, E_PAD - E1), constant_values=N1)

    w1_1, b1_1, w2_1, b2_1 = l1_ffn
    w1_2, b1_2, w2_2, b2_2 = l2_edge_mlp
    w1_6, b1_6, w2_6, b2_6 = l6_edge_mlp
    w1_7, b1_7, w2_7, b2_7 = l7_fbn
    w1p = w1_1[1:4]
    w1r = w1_1[0:1]
    w2b = w1_2[D:D + 3]
    w6b = w1_6[D:D + 3]
    w7a = w1_7[0:D]
    w7b = w1_7[D:D + 3]

    z64 = jnp.zeros((3, D), f32)
    wc = jnp.concatenate([-w1p, z64, w2b, w6b, z64, -w7b], axis=1)
    wc4 = jnp.pad(wc, ((0, 1), (0, 0)))
    centers4 = jnp.pad(centers, ((0, 0), (0, 1)))
    cw_all = _tc_matmul_bias(centers4, wc4, jnp.zeros((1, 6 * D), f32),
                             blk=2000)
    tab1 = cw_all[:, 0:D2]
    cw2 = cw_all[:, D2:D2 + D]
    cw6 = cw_all[:, D2 + D:D2 + 2 * D]
    tab7b = cw_all[:, D2 + 2 * D:D2 + 2 * D + D2]

    pts4 = jnp.concatenate([remission.astype(f32), points.astype(f32)],
                           axis=1)
    w4 = jnp.concatenate([
        jnp.concatenate([w1r, w1p], axis=0),
        jnp.concatenate([jnp.zeros((1, D), f32), w7b], axis=0),
    ], axis=1)
    bq = jnp.concatenate([b1_1, b1_7])[None, :]
    q_cat = _tc_matmul_bias(pts4, w4, bq, blk=2000)

    pre1 = _sc_gather_combine(q_cat, tab1, labels)
    pf128 = _tc_point_mlp(pre1, w2_1, b2_1[None, :])
    t18 = _sc_scatter_add(pf128, labels)
    t1 = t18[:, :_NQ, :D].reshape(N1, D)

    a2, m2 = _tc_prep(t1, cw2, w1_2[0:D], b1_2[None, :])
    pre2 = _sc_two_gather(a2, m2, src_g, dst_g)
    eft2 = _tc_edge_act(pre2, w2_2, b2_2[:, None])
    agg2 = _sc_scatter_max(eft2, dst_s)
    w1o2, b1o2, w2o2, b2o2 = l2_out_mlp
    t2p = _tc_out_mlp(agg2, w1o2, b1o2[:, None], w2o2, b2o2[None, :])
    t2 = t2p[:N1]

    a6, m6 = _tc_prep(t2, cw6, w1_6[0:D], b1_6[None, :])
    pre6 = _sc_two_gather(a6, m6, src_g, dst_g)
    eft6 = _tc_edge_act(pre6, w2_6, b2_6[:, None])
    agg6 = _sc_scatter_max(eft6, dst_s)
    w1o6, b1o6, w2o6, b2o6 = l6_out_mlp
    t6p = _tc_out_mlp(agg6, w1o6, b1o6[:, None], w2o6, b2o6[None, :],
                      res=t2p)
    t6 = t6p[:N1]

    d7 = _tc_d7(t6, w7a, tab7b)
    pre7 = _sc_gather_combine(q_cat, d7, labels)
    return _tc_final(pre7, w2_7, b2_7[None, :], cls_W.astype(f32),
                     cls_b.astype(f32)[None, :])
